# baseline (speedup 1.0000x reference)
import functools

import jax
import jax.numpy as jnp
from jax import lax
from jax.experimental import pallas as pl
from jax.experimental.pallas import tpu as pltpu
from jax.experimental.pallas import tpu_sc as plsc

N = 10000
E = 320000
D = 128
H = 128
L = 20
SLOPE = 0.01

NB = 10
BN = N // NB


def _leaky(x):
    return jnp.where(x >= 0, x, SLOPE * x)


def _dot(a, b):
    return jnp.dot(a, b, preferred_element_type=jnp.float32)


def _full(shape):
    return pl.BlockSpec(shape, lambda i: (0,) * len(shape))


def _nb(w):
    return pl.BlockSpec((BN, w), lambda i: (i, 0))


def _ab(w):
    return pl.BlockSpec((2, BN, w), lambda i: (0, i, 0))


def _blk3(w):
    return pl.BlockSpec((1, BN, w), lambda i: (i, 0, 0))


def _onehot(idx_col):
    io = lax.broadcasted_iota(jnp.int32, (BN, 128), 1)
    return (idx_col == io).astype(jnp.float32)


def _emb_body(c_ref, co_ref, sl_ref, ip_ref, ecat_ref, eco_ref, esl_ref,
              vc_ref, vco_ref, vsl_ref, ipa_ref):
    vc_ref[...] = _dot(_onehot(c_ref[0]), ecat_ref[...])
    vco_ref[...] = _dot(_onehot(co_ref[0]), eco_ref[...])
    vsl_ref[...] = _dot(_onehot(sl_ref[0]), esl_ref[...])
    ipa_ref[...] = jnp.concatenate(
        [ip_ref[...], jnp.ones((BN, 1), jnp.float32),
         jnp.zeros((BN, 95), jnp.float32)], axis=1)


def _emb(inputs_c, inputs_co, inputs_sl, inputs_ip, p):
    ecat = jnp.zeros((128, D), jnp.float32).at[:101].set(p['emb_cat'])
    eco = jnp.zeros((128, D), jnp.float32).at[:92].set(p['emb_co'])
    esl = jnp.zeros((128, D), jnp.float32).at[:6].set(p['emb_sl'])
    return pl.pallas_call(
        _emb_body,
        grid=(NB,),
        in_specs=[_blk3(1), _blk3(1), _blk3(1), _nb(32),
                  _full((128, D)), _full((128, D)), _full((128, D))],
        out_specs=[_nb(D)] * 4,
        out_shape=[jax.ShapeDtypeStruct((N, D), jnp.float32)] * 4,
    )(inputs_c.reshape(NB, BN, 1), inputs_co.reshape(NB, BN, 1),
      inputs_sl.reshape(NB, BN, 1), inputs_ip, ecat, eco, esl)


def _lstm_body(s_ref, t2f_ref, t2b_ref, whf_ref, whb_ref, bf_ref, bb_ref,
               fcw_ref, fcb_ref, vu_ref):
    hf = jnp.zeros((BN, H), jnp.float32)
    cf = jnp.zeros((BN, H), jnp.float32)
    hb = jnp.zeros((BN, H), jnp.float32)
    cb = jnp.zeros((BN, H), jnp.float32)

    def step(h, c, oh, t2, wh, bias):
        g = _dot(oh, t2) + _dot(h, wh) + bias
        i = jax.nn.sigmoid(g[:, :H])
        f = jax.nn.sigmoid(g[:, H:2 * H])
        gg = jnp.tanh(g[:, 2 * H:3 * H])
        o = jax.nn.sigmoid(g[:, 3 * H:])
        c2 = f * c + i * gg
        h2 = o * jnp.tanh(c2)
        return h2, c2

    for t in range(L):
        ohf = _onehot(s_ref[0, :, t:t + 1])
        ohb = _onehot(s_ref[0, :, L - 1 - t:L - t])
        hf, cf = step(hf, cf, ohf, t2f_ref[...], whf_ref[...], bf_ref[...])
        hb, cb = step(hb, cb, ohb, t2b_ref[...], whb_ref[...], bb_ref[...])

    hcat = jnp.concatenate([hf, hb], axis=1)
    vu_ref[...] = _leaky(_dot(hcat, fcw_ref[...]) + fcb_ref[...])


def _lstm(inputs_s, p):
    t2f = _dot(p['emb_url'], p['lstm_Wih_f'])
    t2b = _dot(p['emb_url'], p['lstm_Wih_b'])
    bf = (p['lstm_bih_f'] + p['lstm_bhh_f'])[None, :]
    bb = (p['lstm_bih_b'] + p['lstm_bhh_b'])[None, :]
    return pl.pallas_call(
        _lstm_body,
        grid=(NB,),
        in_specs=[_blk3(L),
                  _full((D, 4 * H)), _full((D, 4 * H)),
                  _full((H, 4 * H)), _full((H, 4 * H)),
                  _full((1, 4 * H)), _full((1, 4 * H)),
                  _full((2 * H, D)), _full((1, D))],
        out_specs=_nb(D),
        out_shape=jax.ShapeDtypeStruct((N, D), jnp.float32),
    )(inputs_s.reshape(NB, BN, L), t2f, t2b, p['lstm_Whh_f'],
      p['lstm_Whh_b'], bf, bb, p['fc_lstm_W'], p['fc_lstm_b'][None, :])


NW = 32
EPW = E // NW
AB = 100
NCH = 5
CB = 20
N2 = 10240
NPS = N2 // 16


@functools.lru_cache(maxsize=None)
def _make_agg(nt):
    mesh = plsc.VectorSubcoreMesh(core_axis_name="c", subcore_axis_name="s")
    scratch = [pltpu.VMEM((CB, AB), jnp.int32),
               pltpu.VMEM((CB, AB), jnp.int32),
               pltpu.SemaphoreType.DMA, pltpu.SemaphoreType.DMA,
               pltpu.VMEM_SHARED((N2, D), jnp.float32),
               pltpu.VMEM((AB, D), jnp.float32),
               pltpu.VMEM((AB, D), jnp.float32)]

    def body(*refs):
        hs = refs[:nt]
        src3, dst3, zeros_h = refs[nt], refs[nt + 1], refs[nt + 2]
        outs = refs[nt + 3:2 * nt + 3]
        src_v, dst_v, sem0, sem1, acc, b0, b1 = refs[2 * nt + 3:]
        c = lax.axis_index("c")
        s = lax.axis_index("s")
        wid = c * 16 + s

        for t in range(nt):
            h = hs[t]
            pltpu.sync_copy(zeros_h.at[pl.ds(s * NPS, NPS)],
                            acc.at[pl.ds(s * NPS, NPS)])
            plsc.subcore_barrier()

            @pl.loop(0, NCH)
            def _(ch, h=h):
                pltpu.sync_copy(src3.at[wid * NCH + ch], src_v)
                pltpu.sync_copy(dst3.at[wid * NCH + ch], dst_v)
                pltpu.async_copy(h.at[src_v.at[0]], b0, sem0)
                pltpu.async_copy(h.at[src_v.at[1]], b1, sem1)

                @pl.loop(0, CB, step=2)
                def _(j, h=h):
                    for bi, (buf, sem) in enumerate(((b0, sem0), (b1, sem1))):
                        k = j + bi
                        pltpu.make_async_copy(h.at[src_v.at[k]], buf,
                                              sem).wait()
                        pltpu.sync_copy(buf, acc.at[dst_v.at[k]], add=True)

                        @pl.when(k + 2 < CB)
                        def _(h=h, buf=buf, sem=sem, k=k):
                            pltpu.async_copy(h.at[src_v.at[k + 2]], buf, sem)

            plsc.subcore_barrier()
            pltpu.sync_copy(acc.at[pl.ds(s * NPS, NPS)], outs[t].at[wid])

    out_type = [jax.ShapeDtypeStruct((NW, NPS, D), jnp.float32)
                for _ in range(nt)]
    return pl.kernel(body, out_type=out_type, mesh=mesh,
                     scratch_types=scratch)


def _agg(h_list, src3, dst3, zeros_h):
    outs = _make_agg(len(h_list))(*h_list, src3, dst3, zeros_h)
    if not isinstance(outs, (list, tuple)):
        outs = (outs,)
    return [o.reshape(2, N2, D) for o in outs]


GB = 128
EP = NW * 10240
GPW = EP // NW
GBAT = GPW // GB


@functools.lru_cache(maxsize=None)
def _make_edge_gather():
    mesh = plsc.VectorSubcoreMesh(core_axis_name="c", subcore_axis_name="s")
    scratch = [pltpu.VMEM((GBAT, GB), jnp.int32),
               pltpu.VMEM((GBAT, GB), jnp.int32)]
    for _ in range(2):
        scratch += [pltpu.VMEM((GB, D), jnp.float32),
                    pltpu.VMEM((GB, D), jnp.float32),
                    pltpu.VMEM((GB, D), jnp.float32)]
    scratch += [pltpu.VMEM((16, 128), jnp.float32)]
    scratch += [pltpu.SemaphoreType.DMA] * 6

    def body(P, Q, A16, se3, de3, gp_o, gq_o, ga_o, se_v, de_v,
             p0, q0, a0, p1, q1, a1, bc, *gs):
        c = lax.axis_index("c")
        s = lax.axis_index("s")
        wid = c * 16 + s
        pltpu.sync_copy(se3.at[wid], se_v)
        pltpu.sync_copy(de3.at[wid], de_v)
        slots = ((p0, q0, a0), (p1, q1, a1))

        def gath(k, sl):
            bp, bq, ba = slots[sl]
            pltpu.async_copy(P.at[se_v.at[k]], bp, gs[3 * sl])
            pltpu.async_copy(Q.at[de_v.at[k]], bq, gs[3 * sl + 1])
            pltpu.async_copy(A16.at[se_v.at[k]], ba, gs[3 * sl + 2])

        gath(0, 0)
        gath(1, 1)

        @pl.loop(0, GBAT, step=2)
        def _(j):
            for sl in range(2):
                k = j + sl
                bp, bq, ba = slots[sl]
                base = wid * GPW + k * GB
                pltpu.make_async_copy(P.at[se_v.at[k]], bp, gs[3 * sl]).wait()
                pltpu.make_async_copy(Q.at[de_v.at[k]], bq,
                                      gs[3 * sl + 1]).wait()
                pltpu.make_async_copy(A16.at[se_v.at[k]], ba,
                                      gs[3 * sl + 2]).wait()
                pltpu.sync_copy(bp, gp_o.at[pl.ds(base, GB)])
                pltpu.sync_copy(bq, gq_o.at[pl.ds(base, GB)])
                for r in range(GB):
                    bc[r // 8, pl.ds((r % 8) * 16, 16)] = ba[r, pl.ds(0, 16)]
                pltpu.sync_copy(bc, ga_o.at[pl.ds(wid * 1280 + k * 16, 16)])

                @pl.when(k + 2 < GBAT)
                def _(k=k, sl=sl):
                    gath(k + 2, sl)

    out_type = [jax.ShapeDtypeStruct((EP, D), jnp.float32),
                jax.ShapeDtypeStruct((EP, D), jnp.float32),
                jax.ShapeDtypeStruct((EP // 8, 128), jnp.float32)]
    return pl.kernel(body, out_type=out_type, mesh=mesh,
                     scratch_types=scratch)


def _edge_gather(P, Q, attn128, se, de):
    pad = jnp.zeros((EP - E,), jnp.int32)
    se3 = jnp.concatenate([se, pad]).reshape(NW, GBAT, GB)
    de3 = jnp.concatenate([de, pad]).reshape(NW, GBAT, GB)
    return _make_edge_gather()(P, Q, attn128, se3, de3)


def _sage_one(h, acc, inv, ws, wn, b):
    mean = (acc[0] + acc[1]) * inv
    return _leaky(_dot(h, ws[...]) + _dot(mean, wn[...]) + b[...])


def _l0a_body(hc_ref, hco_ref, hsl_ref, hip_ref,
              ac_ref, aco_ref, asl_ref, aip_ref,
              wsc, wnc, wsco, wnco, wssl, wnsl, wsip, wnip,
              bc, bco, bsl, bip,
              oc_ref, oco_ref, osl_ref, oip_ref, deg_ref):
    deg = jnp.maximum(aip_ref[0, :, 32:33] + aip_ref[1, :, 32:33], 1.0)
    inv = 1.0 / deg
    deg_ref[...] = deg
    oc_ref[...] = _sage_one(hc_ref[...], ac_ref[...], inv, wsc, wnc, bc)
    oco_ref[...] = _sage_one(hco_ref[...], aco_ref[...], inv, wsco, wnco, bco)
    osl_ref[...] = _sage_one(hsl_ref[...], asl_ref[...], inv, wssl, wnsl, bsl)
    mean_ip = (aip_ref[0, :, :32] + aip_ref[1, :, :32]) * inv
    oip_ref[...] = _leaky(_dot(hip_ref[...], wsip[...]) +
                          _dot(mean_ip, wnip[...]) + bip[...])


def _layer0a(h_c, h_co, h_sl, h_ip, accs, p):
    w = lambda st: (p['sage_%s_0_Wself' % st], p['sage_%s_0_Wneigh' % st])
    b = lambda st: p['sage_%s_0_b' % st][None, :]
    return pl.pallas_call(
        _l0a_body,
        grid=(NB,),
        in_specs=[_nb(D), _nb(D), _nb(D), _nb(32),
                  _ab(D), _ab(D), _ab(D), _ab(D)]
                 + [_full((D, H))] * 6 + [_full((32, H))] * 2
                 + [_full((1, H))] * 4,
        out_specs=[_nb(H)] * 4 + [_nb(1)],
        out_shape=[jax.ShapeDtypeStruct((N, H), jnp.float32)] * 4
                  + [jax.ShapeDtypeStruct((N, 1), jnp.float32)],
    )(h_c, h_co, h_sl, h_ip, *accs,
      *w('c'), *w('co'), *w('sl'), *w('ip'), b('c'), b('co'), b('sl'),
      b('ip'))


def _l0b_body(hs_ref, as_ref, deg_ref, wss, wns, bs, os_ref):
    inv = 1.0 / deg_ref[...]
    os_ref[...] = _sage_one(hs_ref[...], as_ref[...], inv, wss, wns, bs)


def _layer0b(h_s, acc_s, deg, p):
    return pl.pallas_call(
        _l0b_body,
        grid=(NB,),
        in_specs=[_nb(D), _ab(D), _nb(1), _full((D, H)), _full((D, H)),
                  _full((1, H))],
        out_specs=_nb(H),
        out_shape=jax.ShapeDtypeStruct((N, H), jnp.float32),
    )(h_s, acc_s, deg, p['sage_s_0_Wself'], p['sage_s_0_Wneigh'],
      p['sage_s_0_b'][None, :])


def _l1a_body(hc, hco, hsl, hip, ac, aco, asl, aip, deg_ref,
              ws0, ws1, ws2, ws3, wn0, wn1, wn2, wn3, b0, b1, b2, b3,
              alin, aw, ab_,
              lc_ref, lco_ref, lsl_ref, lip_ref, e4_ref):
    inv = 1.0 / deg_ref[...]
    es = []
    for h, a, ws_, wn_, b_, o_ref in (
            (hc, ac, ws0, wn0, b0, lc_ref),
            (hco, aco, ws1, wn1, b1, lco_ref),
            (hsl, asl, ws2, wn2, b2, lsl_ref),
            (hip, aip, ws3, wn3, b3, lip_ref)):
        h2 = _sage_one(h[...], a[...], inv, ws_, wn_, b_)
        lin = _dot(h2, alin[...])
        o_ref[...] = lin
        es.append(_dot(lin, aw[...]) + ab_[...])
    e4_ref[...] = jnp.concatenate(es, axis=1)


def _layer1a(h1, accs1, deg, p):
    sts = ('c', 'co', 'sl', 'ip')
    return pl.pallas_call(
        _l1a_body,
        grid=(NB,),
        in_specs=[_nb(D)] * 4 + [_ab(D)] * 4 + [_nb(1)]
                 + [_full((D, H))] * 8 + [_full((1, H))] * 4
                 + [_full((H, H)), _full((H, 1)), _full((1, 1))],
        out_specs=[_nb(H)] * 4 + [_nb(4)],
        out_shape=[jax.ShapeDtypeStruct((N, H), jnp.float32)] * 4
                  + [jax.ShapeDtypeStruct((N, 4), jnp.float32)],
    )(*h1, *accs1, deg,
      *[p['sage_%s_1_Wself' % st] for st in sts],
      *[p['sage_%s_1_Wneigh' % st] for st in sts],
      *[p['sage_%s_1_b' % st][None, :] for st in sts],
      p['attn_lin_W'], p['attn_W'], p['attn_b'][None, :])


def _l1b_body(hs, as_, deg_ref, lc, lco, lsl, lip, e4_ref,
              wss, wns, bs, alin, aw, ab_, fwt, fwb, fcb,
              p_ref, q_ref, a128_ref):
    inv = 1.0 / deg_ref[...]
    h2 = _sage_one(hs[...], as_[...], inv, wss, wns, bs)
    lin_s = _dot(h2, alin[...])
    e_s = _dot(lin_s, aw[...]) + ab_[...]
    e4 = e4_ref[...]
    lins = [lin_s, lc[...], lco[...], lsl[...], lip[...]]
    es = [e_s] + [e4[:, t:t + 1] for t in range(4)]
    m = es[0]
    for t in range(1, 5):
        m = jnp.maximum(m, es[t])
    exps = [jnp.exp(e - m) for e in es]
    z = exps[0] + exps[1] + exps[2] + exps[3] + exps[4]
    inv_z = 1.0 / z
    a = [ex * inv_z for ex in exps]
    hfin = jnp.zeros((BN, H), jnp.float32)
    for t in range(5):
        hfin = hfin + lins[t] * a[t]
    p_ref[...] = _dot(hfin, fwt[...]) + fcb[...]
    q_ref[...] = _dot(hfin, fwb[...])
    a128_ref[...] = jnp.concatenate(
        a + [jnp.zeros((BN, 123), jnp.float32)], axis=1)


def _layer1b(h1_s, acc1_s, deg, lins, e4, p):
    return pl.pallas_call(
        _l1b_body,
        grid=(NB,),
        in_specs=[_nb(D), _ab(D), _nb(1)] + [_nb(D)] * 4 + [_nb(4)]
                 + [_full((D, H)), _full((D, H)), _full((1, H)),
                    _full((H, H)), _full((H, 1)), _full((1, 1)),
                    _full((H, H)), _full((H, H)), _full((1, H))],
        out_specs=[_nb(H), _nb(H), _nb(H)],
        out_shape=[jax.ShapeDtypeStruct((N, H), jnp.float32)] * 3,
    )(h1_s, acc1_s, deg, *lins, e4,
      p['sage_s_1_Wself'], p['sage_s_1_Wneigh'], p['sage_s_1_b'][None, :],
      p['attn_lin_W'], p['attn_W'], p['attn_b'][None, :],
      p['fc_W'][:H], p['fc_W'][H:], p['fc_b'][None, :])


def _edge_mlp_body(gp_ref, gq_ref, wot, bo, out_ref):
    t = _leaky(gp_ref[...] + gq_ref[...])
    out_ref[...] = lax.dot_general(
        wot[...], t, (((1,), (1,)), ((), ())),
        preferred_element_type=jnp.float32) + bo[...]


def _edge_mlp(gp, gq, p, be):
    ne = gp.shape[0]
    eb = pl.BlockSpec((be, D), lambda i: (i, 0))
    score_t = pl.pallas_call(
        _edge_mlp_body,
        grid=(ne // be,),
        in_specs=[eb, eb, _full((2, D)), _full((2, 1))],
        out_specs=pl.BlockSpec((2, be), lambda i: (0, i)),
        out_shape=jax.ShapeDtypeStruct((2, ne), jnp.float32),
    )(gp, gq, p['fc_out_W'].T, p['fc_out_b'][:, None])
    return score_t


def kernel(inputs_s, inputs_sm, inputs_c, inputs_co, inputs_sl, inputs_ip,
           edge_index, edge_index_sub, params):
    p = params
    vec_cat, vec_co, vec_sl, ip_aug = _emb(
        inputs_c, inputs_co, inputs_sl, inputs_ip, p)

    src3 = edge_index[0].reshape(NW * NCH, CB, AB)
    dst3 = edge_index[1].reshape(NW * NCH, CB, AB)
    zeros_h = jnp.zeros((N2, D), jnp.float32)

    accs0a = _agg([vec_cat, vec_co, vec_sl, ip_aug], src3, dst3, zeros_h)
    vec_url = _lstm(inputs_s, p)
    accs0b = _agg([vec_url], src3, dst3, zeros_h)

    *h1a, deg = _layer0a(vec_cat, vec_co, vec_sl, inputs_ip, accs0a, p)
    h1_s = _layer0b(vec_url, accs0b[0], deg, p)

    accs1a = _agg(h1a, src3, dst3, zeros_h)
    accs1b = _agg([h1_s], src3, dst3, zeros_h)

    *lins, e4 = _layer1a(h1a, accs1a, deg, p)
    P, Q, attn128 = _layer1b(h1_s, accs1b[0], deg, lins, e4, p)

    se, de = edge_index_sub[0], edge_index_sub[1]
    gp, gq, ga = _edge_gather(P, Q, attn128, se, de)
    score_t = _edge_mlp(gp, gq, p, 5120)

    score = score_t[:, :E].T
    a5 = ga.reshape(EP, 16)[:E, :5]
    attn_out = jnp.concatenate([a5[:, None, :], a5[:, None, :]], axis=1)
    return score, attn_out


# speedup vs baseline: 1.1030x; 1.1030x over previous
import functools

import jax
import jax.numpy as jnp
from jax import lax
from jax.experimental import pallas as pl
from jax.experimental.pallas import tpu as pltpu
from jax.experimental.pallas import tpu_sc as plsc

N = 10000
E = 320000
D = 128
H = 128
L = 20
SLOPE = 0.01

NB = 10
BN = N // NB


def _leaky(x):
    return jnp.where(x >= 0, x, SLOPE * x)


def _dot(a, b):
    return jnp.dot(a, b, preferred_element_type=jnp.float32)


def _full(shape):
    return pl.BlockSpec(shape, lambda i: (0,) * len(shape))


def _nb(w):
    return pl.BlockSpec((BN, w), lambda i: (i, 0))


def _ab(w):
    return pl.BlockSpec((2, BN, w), lambda i: (0, i, 0))


def _blk3(w):
    return pl.BlockSpec((1, BN, w), lambda i: (i, 0, 0))


def _onehot(idx_col):
    io = lax.broadcasted_iota(jnp.int32, (BN, 128), 1)
    return (idx_col == io).astype(jnp.float32)


def _emb_body(c_ref, co_ref, sl_ref, ip_ref, ecat_ref, eco_ref, esl_ref,
              vc_ref, vco_ref, vsl_ref, ipa_ref):
    vc_ref[...] = _dot(_onehot(c_ref[0]), ecat_ref[...])
    vco_ref[...] = _dot(_onehot(co_ref[0]), eco_ref[...])
    vsl_ref[...] = _dot(_onehot(sl_ref[0]), esl_ref[...])
    ipa_ref[...] = jnp.concatenate(
        [ip_ref[...], jnp.ones((BN, 1), jnp.float32),
         jnp.zeros((BN, 95), jnp.float32)], axis=1)


def _emb(inputs_c, inputs_co, inputs_sl, inputs_ip, p):
    ecat = jnp.zeros((128, D), jnp.float32).at[:101].set(p['emb_cat'])
    eco = jnp.zeros((128, D), jnp.float32).at[:92].set(p['emb_co'])
    esl = jnp.zeros((128, D), jnp.float32).at[:6].set(p['emb_sl'])
    return pl.pallas_call(
        _emb_body,
        grid=(NB,),
        in_specs=[_blk3(1), _blk3(1), _blk3(1), _nb(32),
                  _full((128, D)), _full((128, D)), _full((128, D))],
        out_specs=[_nb(D)] * 4,
        out_shape=[jax.ShapeDtypeStruct((N, D), jnp.float32)] * 4,
    )(inputs_c.reshape(NB, BN, 1), inputs_co.reshape(NB, BN, 1),
      inputs_sl.reshape(NB, BN, 1), inputs_ip, ecat, eco, esl)


def _lstm_body(s_ref, t2f_ref, t2b_ref, whf_ref, whb_ref, bf_ref, bb_ref,
               fcw_ref, fcb_ref, vu_ref):
    hf = jnp.zeros((BN, H), jnp.float32)
    cf = jnp.zeros((BN, H), jnp.float32)
    hb = jnp.zeros((BN, H), jnp.float32)
    cb = jnp.zeros((BN, H), jnp.float32)

    def step(h, c, oh, t2, wh, bias):
        g = _dot(oh, t2) + _dot(h, wh) + bias
        i = jax.nn.sigmoid(g[:, :H])
        f = jax.nn.sigmoid(g[:, H:2 * H])
        gg = jnp.tanh(g[:, 2 * H:3 * H])
        o = jax.nn.sigmoid(g[:, 3 * H:])
        c2 = f * c + i * gg
        h2 = o * jnp.tanh(c2)
        return h2, c2

    for t in range(L):
        ohf = _onehot(s_ref[0, :, t:t + 1])
        ohb = _onehot(s_ref[0, :, L - 1 - t:L - t])
        hf, cf = step(hf, cf, ohf, t2f_ref[...], whf_ref[...], bf_ref[...])
        hb, cb = step(hb, cb, ohb, t2b_ref[...], whb_ref[...], bb_ref[...])

    hcat = jnp.concatenate([hf, hb], axis=1)
    vu_ref[...] = _leaky(_dot(hcat, fcw_ref[...]) + fcb_ref[...])


def _lstm(inputs_s, p):
    t2f = _dot(p['emb_url'], p['lstm_Wih_f'])
    t2b = _dot(p['emb_url'], p['lstm_Wih_b'])
    bf = (p['lstm_bih_f'] + p['lstm_bhh_f'])[None, :]
    bb = (p['lstm_bih_b'] + p['lstm_bhh_b'])[None, :]
    return pl.pallas_call(
        _lstm_body,
        grid=(NB,),
        in_specs=[_blk3(L),
                  _full((D, 4 * H)), _full((D, 4 * H)),
                  _full((H, 4 * H)), _full((H, 4 * H)),
                  _full((1, 4 * H)), _full((1, 4 * H)),
                  _full((2 * H, D)), _full((1, D))],
        out_specs=_nb(D),
        out_shape=jax.ShapeDtypeStruct((N, D), jnp.float32),
    )(inputs_s.reshape(NB, BN, L), t2f, t2b, p['lstm_Whh_f'],
      p['lstm_Whh_b'], bf, bb, p['fc_lstm_W'], p['fc_lstm_b'][None, :])


NW = 32
EPW = E // NW
AB = 100
NCH = 5
CB = 20
N2 = 10240
NPS = N2 // 16


@functools.lru_cache(maxsize=None)
def _make_agg(nt):
    mesh = plsc.VectorSubcoreMesh(core_axis_name="c", subcore_axis_name="s")
    scratch = [pltpu.VMEM((CB, AB), jnp.int32),
               pltpu.VMEM((CB, AB), jnp.int32),
               pltpu.SemaphoreType.DMA, pltpu.SemaphoreType.DMA,
               pltpu.VMEM_SHARED((N2, D), jnp.float32),
               pltpu.VMEM((AB, D), jnp.float32),
               pltpu.VMEM((AB, D), jnp.float32)]

    def body(*refs):
        hs = refs[:nt]
        src3, dst3, zeros_h = refs[nt], refs[nt + 1], refs[nt + 2]
        outs = refs[nt + 3:2 * nt + 3]
        src_v, dst_v, sem0, sem1, acc, b0, b1 = refs[2 * nt + 3:]
        c = lax.axis_index("c")
        s = lax.axis_index("s")
        wid = c * 16 + s

        for t in range(nt):
            h = hs[t]
            pltpu.sync_copy(zeros_h.at[pl.ds(s * NPS, NPS)],
                            acc.at[pl.ds(s * NPS, NPS)])
            plsc.subcore_barrier()

            @pl.loop(0, NCH)
            def _(ch, h=h):
                pltpu.sync_copy(src3.at[wid * NCH + ch], src_v)
                pltpu.sync_copy(dst3.at[wid * NCH + ch], dst_v)
                pltpu.async_copy(h.at[src_v.at[0]], b0, sem0)
                pltpu.async_copy(h.at[src_v.at[1]], b1, sem1)

                @pl.loop(0, CB, step=2)
                def _(j, h=h):
                    for bi, (buf, sem) in enumerate(((b0, sem0), (b1, sem1))):
                        k = j + bi
                        pltpu.make_async_copy(h.at[src_v.at[k]], buf,
                                              sem).wait()
                        pltpu.sync_copy(buf, acc.at[dst_v.at[k]], add=True)

                        @pl.when(k + 2 < CB)
                        def _(h=h, buf=buf, sem=sem, k=k):
                            pltpu.async_copy(h.at[src_v.at[k + 2]], buf, sem)

            plsc.subcore_barrier()
            pltpu.sync_copy(acc.at[pl.ds(s * NPS, NPS)], outs[t].at[wid])

    out_type = [jax.ShapeDtypeStruct((NW, NPS, D), jnp.float32)
                for _ in range(nt)]
    return pl.kernel(body, out_type=out_type, mesh=mesh,
                     scratch_types=scratch)


def _agg(h_list, src3, dst3, zeros_h):
    outs = _make_agg(len(h_list))(*h_list, src3, dst3, zeros_h)
    if not isinstance(outs, (list, tuple)):
        outs = (outs,)
    return [o.reshape(2, N2, D) for o in outs]


GB = 128
EP = NW * 10240
GPW = EP // NW
GBAT = GPW // GB


@functools.lru_cache(maxsize=None)
def _make_edge_gather():
    mesh = plsc.VectorSubcoreMesh(core_axis_name="c", subcore_axis_name="s")
    scratch = [pltpu.VMEM((GBAT, GB), jnp.int32),
               pltpu.VMEM((GBAT, GB), jnp.int32)]
    for _ in range(2):
        scratch += [pltpu.VMEM((GB, D), jnp.float32),
                    pltpu.VMEM((GB, D), jnp.float32),
                    pltpu.VMEM((GB, D), jnp.float32)]
    scratch += [pltpu.SemaphoreType.DMA] * 6

    def body(P, Q, A16, se3, de3, gp_o, gq_o, ga_o, se_v, de_v,
             p0, q0, a0, p1, q1, a1, *gs):
        c = lax.axis_index("c")
        s = lax.axis_index("s")
        wid = c * 16 + s
        pltpu.sync_copy(se3.at[wid], se_v)
        pltpu.sync_copy(de3.at[wid], de_v)
        slots = ((p0, q0, a0), (p1, q1, a1))

        def gath(k, sl):
            bp, bq, ba = slots[sl]
            pltpu.async_copy(P.at[se_v.at[k]], bp, gs[3 * sl])
            pltpu.async_copy(Q.at[de_v.at[k]], bq, gs[3 * sl + 1])
            pltpu.async_copy(A16.at[se_v.at[k]], ba, gs[3 * sl + 2])

        gath(0, 0)
        gath(1, 1)

        @pl.loop(0, GBAT, step=2)
        def _(j):
            for sl in range(2):
                k = j + sl
                bp, bq, ba = slots[sl]
                base = wid * GPW + k * GB
                pltpu.make_async_copy(P.at[se_v.at[k]], bp, gs[3 * sl]).wait()
                pltpu.make_async_copy(Q.at[de_v.at[k]], bq,
                                      gs[3 * sl + 1]).wait()
                pltpu.make_async_copy(A16.at[se_v.at[k]], ba,
                                      gs[3 * sl + 2]).wait()
                pltpu.sync_copy(bp, gp_o.at[pl.ds(base, GB)])
                pltpu.sync_copy(bq, gq_o.at[pl.ds(base, GB)])
                pltpu.sync_copy(ba, ga_o.at[pl.ds(base, GB)])

                @pl.when(k + 2 < GBAT)
                def _(k=k, sl=sl):
                    gath(k + 2, sl)

    out_type = [jax.ShapeDtypeStruct((EP, D), jnp.float32),
                jax.ShapeDtypeStruct((EP, D), jnp.float32),
                jax.ShapeDtypeStruct((EP, D), jnp.float32)]
    return pl.kernel(body, out_type=out_type, mesh=mesh,
                     scratch_types=scratch)


def _edge_gather(P, Q, attn128, se, de):
    pad = jnp.zeros((EP - E,), jnp.int32)
    se3 = jnp.concatenate([se, pad]).reshape(NW, GBAT, GB)
    de3 = jnp.concatenate([de, pad]).reshape(NW, GBAT, GB)
    return _make_edge_gather()(P, Q, attn128, se3, de3)


def _sage_one(h, acc, inv, ws, wn, b):
    mean = (acc[0] + acc[1]) * inv
    return _leaky(_dot(h, ws[...]) + _dot(mean, wn[...]) + b[...])


def _l0a_body(hc_ref, hco_ref, hsl_ref, hip_ref,
              ac_ref, aco_ref, asl_ref, aip_ref,
              wsc, wnc, wsco, wnco, wssl, wnsl, wsip, wnip,
              bc, bco, bsl, bip,
              oc_ref, oco_ref, osl_ref, oip_ref, deg_ref):
    deg = jnp.maximum(aip_ref[0, :, 32:33] + aip_ref[1, :, 32:33], 1.0)
    inv = 1.0 / deg
    deg_ref[...] = deg
    oc_ref[...] = _sage_one(hc_ref[...], ac_ref[...], inv, wsc, wnc, bc)
    oco_ref[...] = _sage_one(hco_ref[...], aco_ref[...], inv, wsco, wnco, bco)
    osl_ref[...] = _sage_one(hsl_ref[...], asl_ref[...], inv, wssl, wnsl, bsl)
    mean_ip = (aip_ref[0, :, :32] + aip_ref[1, :, :32]) * inv
    oip_ref[...] = _leaky(_dot(hip_ref[...], wsip[...]) +
                          _dot(mean_ip, wnip[...]) + bip[...])


def _layer0a(h_c, h_co, h_sl, h_ip, accs, p):
    w = lambda st: (p['sage_%s_0_Wself' % st], p['sage_%s_0_Wneigh' % st])
    b = lambda st: p['sage_%s_0_b' % st][None, :]
    return pl.pallas_call(
        _l0a_body,
        grid=(NB,),
        in_specs=[_nb(D), _nb(D), _nb(D), _nb(32),
                  _ab(D), _ab(D), _ab(D), _ab(D)]
                 + [_full((D, H))] * 6 + [_full((32, H))] * 2
                 + [_full((1, H))] * 4,
        out_specs=[_nb(H)] * 4 + [_nb(1)],
        out_shape=[jax.ShapeDtypeStruct((N, H), jnp.float32)] * 4
                  + [jax.ShapeDtypeStruct((N, 1), jnp.float32)],
    )(h_c, h_co, h_sl, h_ip, *accs,
      *w('c'), *w('co'), *w('sl'), *w('ip'), b('c'), b('co'), b('sl'),
      b('ip'))


def _l0b_body(hs_ref, as_ref, deg_ref, wss, wns, bs, os_ref):
    inv = 1.0 / deg_ref[...]
    os_ref[...] = _sage_one(hs_ref[...], as_ref[...], inv, wss, wns, bs)


def _layer0b(h_s, acc_s, deg, p):
    return pl.pallas_call(
        _l0b_body,
        grid=(NB,),
        in_specs=[_nb(D), _ab(D), _nb(1), _full((D, H)), _full((D, H)),
                  _full((1, H))],
        out_specs=_nb(H),
        out_shape=jax.ShapeDtypeStruct((N, H), jnp.float32),
    )(h_s, acc_s, deg, p['sage_s_0_Wself'], p['sage_s_0_Wneigh'],
      p['sage_s_0_b'][None, :])


def _l1a_body(hc, hco, hsl, hip, ac, aco, asl, aip, deg_ref,
              ws0, ws1, ws2, ws3, wn0, wn1, wn2, wn3, b0, b1, b2, b3,
              alin, aw, ab_,
              lc_ref, lco_ref, lsl_ref, lip_ref, e4_ref):
    inv = 1.0 / deg_ref[...]
    es = []
    for h, a, ws_, wn_, b_, o_ref in (
            (hc, ac, ws0, wn0, b0, lc_ref),
            (hco, aco, ws1, wn1, b1, lco_ref),
            (hsl, asl, ws2, wn2, b2, lsl_ref),
            (hip, aip, ws3, wn3, b3, lip_ref)):
        h2 = _sage_one(h[...], a[...], inv, ws_, wn_, b_)
        lin = _dot(h2, alin[...])
        o_ref[...] = lin
        es.append(_dot(lin, aw[...]) + ab_[...])
    e4_ref[...] = jnp.concatenate(es, axis=1)


def _layer1a(h1, accs1, deg, p):
    sts = ('c', 'co', 'sl', 'ip')
    return pl.pallas_call(
        _l1a_body,
        grid=(NB,),
        in_specs=[_nb(D)] * 4 + [_ab(D)] * 4 + [_nb(1)]
                 + [_full((D, H))] * 8 + [_full((1, H))] * 4
                 + [_full((H, H)), _full((H, 1)), _full((1, 1))],
        out_specs=[_nb(H)] * 4 + [_nb(4)],
        out_shape=[jax.ShapeDtypeStruct((N, H), jnp.float32)] * 4
                  + [jax.ShapeDtypeStruct((N, 4), jnp.float32)],
    )(*h1, *accs1, deg,
      *[p['sage_%s_1_Wself' % st] for st in sts],
      *[p['sage_%s_1_Wneigh' % st] for st in sts],
      *[p['sage_%s_1_b' % st][None, :] for st in sts],
      p['attn_lin_W'], p['attn_W'], p['attn_b'][None, :])


def _l1b_body(hs, as_, deg_ref, lc, lco, lsl, lip, e4_ref,
              wss, wns, bs, alin, aw, ab_, fwt, fwb, fcb,
              p_ref, q_ref, a128_ref):
    inv = 1.0 / deg_ref[...]
    h2 = _sage_one(hs[...], as_[...], inv, wss, wns, bs)
    lin_s = _dot(h2, alin[...])
    e_s = _dot(lin_s, aw[...]) + ab_[...]
    e4 = e4_ref[...]
    lins = [lin_s, lc[...], lco[...], lsl[...], lip[...]]
    es = [e_s] + [e4[:, t:t + 1] for t in range(4)]
    m = es[0]
    for t in range(1, 5):
        m = jnp.maximum(m, es[t])
    exps = [jnp.exp(e - m) for e in es]
    z = exps[0] + exps[1] + exps[2] + exps[3] + exps[4]
    inv_z = 1.0 / z
    a = [ex * inv_z for ex in exps]
    hfin = jnp.zeros((BN, H), jnp.float32)
    for t in range(5):
        hfin = hfin + lins[t] * a[t]
    p_ref[...] = _dot(hfin, fwt[...]) + fcb[...]
    q_ref[...] = _dot(hfin, fwb[...])
    a128_ref[...] = jnp.concatenate(
        a + [jnp.zeros((BN, 123), jnp.float32)], axis=1)


def _layer1b(h1_s, acc1_s, deg, lins, e4, p):
    return pl.pallas_call(
        _l1b_body,
        grid=(NB,),
        in_specs=[_nb(D), _ab(D), _nb(1)] + [_nb(D)] * 4 + [_nb(4)]
                 + [_full((D, H)), _full((D, H)), _full((1, H)),
                    _full((H, H)), _full((H, 1)), _full((1, 1)),
                    _full((H, H)), _full((H, H)), _full((1, H))],
        out_specs=[_nb(H), _nb(H), _nb(H)],
        out_shape=[jax.ShapeDtypeStruct((N, H), jnp.float32)] * 3,
    )(h1_s, acc1_s, deg, *lins, e4,
      p['sage_s_1_Wself'], p['sage_s_1_Wneigh'], p['sage_s_1_b'][None, :],
      p['attn_lin_W'], p['attn_W'], p['attn_b'][None, :],
      p['fc_W'][:H], p['fc_W'][H:], p['fc_b'][None, :])


def _edge_mlp_body(gp_ref, gq_ref, wot, bo, out_ref):
    t = _leaky(gp_ref[...] + gq_ref[...])
    out_ref[...] = lax.dot_general(
        wot[...], t, (((1,), (1,)), ((), ())),
        preferred_element_type=jnp.float32) + bo[...]


def _edge_mlp(gp, gq, p, be):
    ne = gp.shape[0]
    eb = pl.BlockSpec((be, D), lambda i: (i, 0))
    score_t = pl.pallas_call(
        _edge_mlp_body,
        grid=(ne // be,),
        in_specs=[eb, eb, _full((2, D)), _full((2, 1))],
        out_specs=pl.BlockSpec((2, be), lambda i: (0, i)),
        out_shape=jax.ShapeDtypeStruct((2, ne), jnp.float32),
    )(gp, gq, p['fc_out_W'].T, p['fc_out_b'][:, None])
    return score_t


def kernel(inputs_s, inputs_sm, inputs_c, inputs_co, inputs_sl, inputs_ip,
           edge_index, edge_index_sub, params):
    p = params
    vec_cat, vec_co, vec_sl, ip_aug = _emb(
        inputs_c, inputs_co, inputs_sl, inputs_ip, p)

    src3 = edge_index[0].reshape(NW * NCH, CB, AB)
    dst3 = edge_index[1].reshape(NW * NCH, CB, AB)
    zeros_h = jnp.zeros((N2, D), jnp.float32)

    accs0a = _agg([vec_cat, vec_co, vec_sl, ip_aug], src3, dst3, zeros_h)
    vec_url = _lstm(inputs_s, p)
    accs0b = _agg([vec_url], src3, dst3, zeros_h)

    *h1a, deg = _layer0a(vec_cat, vec_co, vec_sl, inputs_ip, accs0a, p)
    h1_s = _layer0b(vec_url, accs0b[0], deg, p)

    accs1a = _agg(h1a, src3, dst3, zeros_h)
    accs1b = _agg([h1_s], src3, dst3, zeros_h)

    *lins, e4 = _layer1a(h1a, accs1a, deg, p)
    P, Q, attn128 = _layer1b(h1_s, accs1b[0], deg, lins, e4, p)

    se, de = edge_index_sub[0], edge_index_sub[1]
    gp, gq, ga = _edge_gather(P, Q, attn128, se, de)
    score_t = _edge_mlp(gp, gq, p, 5120)

    score = score_t[:, :E].T
    a5 = ga[:E, :5]
    attn_out = jnp.concatenate([a5[:, None, :], a5[:, None, :]], axis=1)
    return score, attn_out


# speedup vs baseline: 1.2951x; 1.1741x over previous
import functools

import jax
import jax.numpy as jnp
from jax import lax
from jax.experimental import pallas as pl
from jax.experimental.pallas import tpu as pltpu
from jax.experimental.pallas import tpu_sc as plsc

N = 10000
E = 320000
D = 128
H = 128
L = 20
SLOPE = 0.01

NB = 10
BN = N // NB


def _leaky(x):
    return jnp.where(x >= 0, x, SLOPE * x)


def _dot(a, b):
    return jnp.dot(a, b, preferred_element_type=jnp.float32)


def _full(shape):
    return pl.BlockSpec(shape, lambda i: (0,) * len(shape))


def _nb(w):
    return pl.BlockSpec((BN, w), lambda i: (i, 0))


def _ab(w):
    return pl.BlockSpec((2, BN, w), lambda i: (0, i, 0))


def _blk3(w):
    return pl.BlockSpec((1, BN, w), lambda i: (i, 0, 0))


def _onehot(idx_col):
    io = lax.broadcasted_iota(jnp.int32, (BN, 128), 1)
    return (idx_col == io).astype(jnp.float32)


def _emb_body(c_ref, co_ref, sl_ref, ip_ref, ecat_ref, eco_ref, esl_ref,
              vc_ref, vco_ref, vsl_ref, ipa_ref):
    vc_ref[...] = _dot(_onehot(c_ref[0]), ecat_ref[...])
    vco_ref[...] = _dot(_onehot(co_ref[0]), eco_ref[...])
    vsl_ref[...] = _dot(_onehot(sl_ref[0]), esl_ref[...])
    ipa_ref[...] = jnp.concatenate(
        [ip_ref[...], jnp.ones((BN, 1), jnp.float32),
         jnp.zeros((BN, 95), jnp.float32)], axis=1)


def _emb(inputs_c, inputs_co, inputs_sl, inputs_ip, p):
    ecat = jnp.zeros((128, D), jnp.float32).at[:101].set(p['emb_cat'])
    eco = jnp.zeros((128, D), jnp.float32).at[:92].set(p['emb_co'])
    esl = jnp.zeros((128, D), jnp.float32).at[:6].set(p['emb_sl'])
    return pl.pallas_call(
        _emb_body,
        grid=(NB,),
        in_specs=[_blk3(1), _blk3(1), _blk3(1), _nb(32),
                  _full((128, D)), _full((128, D)), _full((128, D))],
        out_specs=[_nb(D)] * 4,
        out_shape=[jax.ShapeDtypeStruct((N, D), jnp.float32)] * 4,
    )(inputs_c.reshape(NB, BN, 1), inputs_co.reshape(NB, BN, 1),
      inputs_sl.reshape(NB, BN, 1), inputs_ip, ecat, eco, esl)


def _lstm_body(s_ref, t2f_ref, t2b_ref, whf_ref, whb_ref, bf_ref, bb_ref,
               fcw_ref, fcb_ref, vu_ref):
    hf = jnp.zeros((BN, H), jnp.float32)
    cf = jnp.zeros((BN, H), jnp.float32)
    hb = jnp.zeros((BN, H), jnp.float32)
    cb = jnp.zeros((BN, H), jnp.float32)

    def step(h, c, oh, t2, wh, bias):
        g = _dot(oh, t2) + _dot(h, wh) + bias
        i = jax.nn.sigmoid(g[:, :H])
        f = jax.nn.sigmoid(g[:, H:2 * H])
        gg = jnp.tanh(g[:, 2 * H:3 * H])
        o = jax.nn.sigmoid(g[:, 3 * H:])
        c2 = f * c + i * gg
        h2 = o * jnp.tanh(c2)
        return h2, c2

    for t in range(L):
        ohf = _onehot(s_ref[0, :, t:t + 1])
        ohb = _onehot(s_ref[0, :, L - 1 - t:L - t])
        hf, cf = step(hf, cf, ohf, t2f_ref[...], whf_ref[...], bf_ref[...])
        hb, cb = step(hb, cb, ohb, t2b_ref[...], whb_ref[...], bb_ref[...])

    hcat = jnp.concatenate([hf, hb], axis=1)
    vu_ref[...] = _leaky(_dot(hcat, fcw_ref[...]) + fcb_ref[...])


def _lstm(inputs_s, p):
    t2f = _dot(p['emb_url'], p['lstm_Wih_f'])
    t2b = _dot(p['emb_url'], p['lstm_Wih_b'])
    bf = (p['lstm_bih_f'] + p['lstm_bhh_f'])[None, :]
    bb = (p['lstm_bih_b'] + p['lstm_bhh_b'])[None, :]
    return pl.pallas_call(
        _lstm_body,
        grid=(NB,),
        in_specs=[_blk3(L),
                  _full((D, 4 * H)), _full((D, 4 * H)),
                  _full((H, 4 * H)), _full((H, 4 * H)),
                  _full((1, 4 * H)), _full((1, 4 * H)),
                  _full((2 * H, D)), _full((1, D))],
        out_specs=_nb(D),
        out_shape=jax.ShapeDtypeStruct((N, D), jnp.float32),
    )(inputs_s.reshape(NB, BN, L), t2f, t2b, p['lstm_Whh_f'],
      p['lstm_Whh_b'], bf, bb, p['fc_lstm_W'], p['fc_lstm_b'][None, :])


NW = 32
EPW = E // NW
AB = 100
NCH = 5
CB = 20
N2 = 10240
NPS = N2 // 16


@functools.lru_cache(maxsize=None)
def _make_agg(nt):
    mesh = plsc.VectorSubcoreMesh(core_axis_name="c", subcore_axis_name="s")
    scratch = [pltpu.VMEM((CB, AB), jnp.int32),
               pltpu.VMEM((CB, AB), jnp.int32),
               pltpu.SemaphoreType.DMA, pltpu.SemaphoreType.DMA,
               pltpu.VMEM_SHARED((N2, D), jnp.float32),
               pltpu.VMEM((AB, D), jnp.float32),
               pltpu.VMEM((AB, D), jnp.float32)]

    def body(*refs):
        hs = refs[:nt]
        src3, dst3, zeros_h = refs[nt], refs[nt + 1], refs[nt + 2]
        outs = refs[nt + 3:2 * nt + 3]
        src_v, dst_v, sem0, sem1, acc, b0, b1 = refs[2 * nt + 3:]
        c = lax.axis_index("c")
        s = lax.axis_index("s")
        wid = c * 16 + s

        for t in range(nt):
            h = hs[t]
            pltpu.sync_copy(zeros_h.at[pl.ds(s * NPS, NPS)],
                            acc.at[pl.ds(s * NPS, NPS)])
            plsc.subcore_barrier()

            @pl.loop(0, NCH)
            def _(ch, h=h):
                pltpu.sync_copy(src3.at[wid * NCH + ch], src_v)
                pltpu.sync_copy(dst3.at[wid * NCH + ch], dst_v)
                pltpu.async_copy(h.at[src_v.at[0]], b0, sem0)
                pltpu.async_copy(h.at[src_v.at[1]], b1, sem1)

                @pl.loop(0, CB, step=2)
                def _(j, h=h):
                    for bi, (buf, sem) in enumerate(((b0, sem0), (b1, sem1))):
                        k = j + bi
                        pltpu.make_async_copy(h.at[src_v.at[k]], buf,
                                              sem).wait()
                        pltpu.sync_copy(buf, acc.at[dst_v.at[k]], add=True)

                        @pl.when(k + 2 < CB)
                        def _(h=h, buf=buf, sem=sem, k=k):
                            pltpu.async_copy(h.at[src_v.at[k + 2]], buf, sem)

            plsc.subcore_barrier()
            pltpu.sync_copy(acc.at[pl.ds(s * NPS, NPS)], outs[t].at[wid])

    out_type = [jax.ShapeDtypeStruct((NW, NPS, D), jnp.float32)
                for _ in range(nt)]
    return pl.kernel(body, out_type=out_type, mesh=mesh,
                     scratch_types=scratch)


def _agg(h_list, src3, dst3, zeros_h):
    outs = _make_agg(len(h_list))(*h_list, src3, dst3, zeros_h)
    if not isinstance(outs, (list, tuple)):
        outs = (outs,)
    return [o.reshape(2, N2, D) for o in outs]


GB = 80


@functools.lru_cache(maxsize=None)
def _make_edge_gather(nbat):
    epw = nbat * GB
    mesh = plsc.VectorSubcoreMesh(core_axis_name="c", subcore_axis_name="s")
    scratch = [pltpu.VMEM((nbat, GB), jnp.int32),
               pltpu.VMEM((nbat, GB), jnp.int32)]
    for _ in range(2):
        scratch += [pltpu.VMEM((GB, D), jnp.float32),
                    pltpu.VMEM((GB, D), jnp.float32),
                    pltpu.VMEM((GB, D), jnp.float32)]
    scratch += [pltpu.SemaphoreType.DMA] * 6

    def body(P, Q, A16, se3, de3, gp_o, gq_o, ga_o, se_v, de_v,
             p0, q0, a0, p1, q1, a1, *gs):
        c = lax.axis_index("c")
        s = lax.axis_index("s")
        wid = c * 16 + s
        pltpu.sync_copy(se3.at[wid], se_v)
        pltpu.sync_copy(de3.at[wid], de_v)
        slots = ((p0, q0, a0), (p1, q1, a1))

        def gath(k, sl):
            bp, bq, ba = slots[sl]
            pltpu.async_copy(P.at[se_v.at[k]], bp, gs[3 * sl])
            pltpu.async_copy(Q.at[de_v.at[k]], bq, gs[3 * sl + 1])
            pltpu.async_copy(A16.at[se_v.at[k]], ba, gs[3 * sl + 2])

        def drain_and_write(k, sl, prefetch):
            bp, bq, ba = slots[sl]
            base = wid * epw + k * GB
            pltpu.make_async_copy(P.at[se_v.at[k]], bp, gs[3 * sl]).wait()
            pltpu.make_async_copy(Q.at[de_v.at[k]], bq,
                                  gs[3 * sl + 1]).wait()
            pltpu.make_async_copy(A16.at[se_v.at[k]], ba,
                                  gs[3 * sl + 2]).wait()
            pltpu.sync_copy(bp, gp_o.at[pl.ds(base, GB)])
            pltpu.sync_copy(bq, gq_o.at[pl.ds(base, GB)])
            pltpu.sync_copy(ba, ga_o.at[pl.ds(base, GB)])
            if prefetch:
                @pl.when(k + 2 < nbat)
                def _(k=k, sl=sl):
                    gath(k + 2, sl)

        gath(0, 0)
        gath(1, 1)

        if nbat % 2 == 0:
            @pl.loop(0, nbat, step=2)
            def _(j):
                for sl in range(2):
                    drain_and_write(j + sl, sl, True)
        else:
            @pl.loop(0, nbat - 1, step=2)
            def _(j):
                for sl in range(2):
                    drain_and_write(j + sl, sl, True)

            drain_and_write(nbat - 1, (nbat - 1) % 2, False)

    out_type = [jax.ShapeDtypeStruct((NW * epw, D), jnp.float32)] * 3
    return pl.kernel(body, out_type=out_type, mesh=mesh,
                     scratch_types=scratch)


def _edge_gather_half(P, Q, attn128, se, de, nbat):
    se3 = se.reshape(NW, nbat, GB)
    de3 = de.reshape(NW, nbat, GB)
    return _make_edge_gather(nbat)(P, Q, attn128, se3, de3)


E1 = 163840
NB1 = E1 // (NW * GB)
NB2 = (E - E1) // (NW * GB)


def _sage_one(h, acc, inv, ws, wn, b):
    mean = (acc[0] + acc[1]) * inv
    return _leaky(_dot(h, ws[...]) + _dot(mean, wn[...]) + b[...])


def _l0a_body(hc_ref, hco_ref, hsl_ref, hip_ref,
              ac_ref, aco_ref, asl_ref, aip_ref,
              wsc, wnc, wsco, wnco, wssl, wnsl, wsip, wnip,
              bc, bco, bsl, bip,
              oc_ref, oco_ref, osl_ref, oip_ref, deg_ref):
    deg = jnp.maximum(aip_ref[0, :, 32:33] + aip_ref[1, :, 32:33], 1.0)
    inv = 1.0 / deg
    deg_ref[...] = deg
    oc_ref[...] = _sage_one(hc_ref[...], ac_ref[...], inv, wsc, wnc, bc)
    oco_ref[...] = _sage_one(hco_ref[...], aco_ref[...], inv, wsco, wnco, bco)
    osl_ref[...] = _sage_one(hsl_ref[...], asl_ref[...], inv, wssl, wnsl, bsl)
    mean_ip = (aip_ref[0, :, :32] + aip_ref[1, :, :32]) * inv
    oip_ref[...] = _leaky(_dot(hip_ref[...], wsip[...]) +
                          _dot(mean_ip, wnip[...]) + bip[...])


def _layer0a(h_c, h_co, h_sl, h_ip, accs, p):
    w = lambda st: (p['sage_%s_0_Wself' % st], p['sage_%s_0_Wneigh' % st])
    b = lambda st: p['sage_%s_0_b' % st][None, :]
    return pl.pallas_call(
        _l0a_body,
        grid=(NB,),
        in_specs=[_nb(D), _nb(D), _nb(D), _nb(32),
                  _ab(D), _ab(D), _ab(D), _ab(D)]
                 + [_full((D, H))] * 6 + [_full((32, H))] * 2
                 + [_full((1, H))] * 4,
        out_specs=[_nb(H)] * 4 + [_nb(1)],
        out_shape=[jax.ShapeDtypeStruct((N, H), jnp.float32)] * 4
                  + [jax.ShapeDtypeStruct((N, 1), jnp.float32)],
    )(h_c, h_co, h_sl, h_ip, *accs,
      *w('c'), *w('co'), *w('sl'), *w('ip'), b('c'), b('co'), b('sl'),
      b('ip'))


def _l0b_body(hs_ref, as_ref, deg_ref, wss, wns, bs, os_ref):
    inv = 1.0 / deg_ref[...]
    os_ref[...] = _sage_one(hs_ref[...], as_ref[...], inv, wss, wns, bs)


def _layer0b(h_s, acc_s, deg, p):
    return pl.pallas_call(
        _l0b_body,
        grid=(NB,),
        in_specs=[_nb(D), _ab(D), _nb(1), _full((D, H)), _full((D, H)),
                  _full((1, H))],
        out_specs=_nb(H),
        out_shape=jax.ShapeDtypeStruct((N, H), jnp.float32),
    )(h_s, acc_s, deg, p['sage_s_0_Wself'], p['sage_s_0_Wneigh'],
      p['sage_s_0_b'][None, :])


def _l1a_body(hc, hco, hsl, hip, ac, aco, asl, aip, deg_ref,
              ws0, ws1, ws2, ws3, wn0, wn1, wn2, wn3, b0, b1, b2, b3,
              alin, aw, ab_,
              lc_ref, lco_ref, lsl_ref, lip_ref, e4_ref):
    inv = 1.0 / deg_ref[...]
    es = []
    for h, a, ws_, wn_, b_, o_ref in (
            (hc, ac, ws0, wn0, b0, lc_ref),
            (hco, aco, ws1, wn1, b1, lco_ref),
            (hsl, asl, ws2, wn2, b2, lsl_ref),
            (hip, aip, ws3, wn3, b3, lip_ref)):
        h2 = _sage_one(h[...], a[...], inv, ws_, wn_, b_)
        lin = _dot(h2, alin[...])
        o_ref[...] = lin
        es.append(_dot(lin, aw[...]) + ab_[...])
    e4_ref[...] = jnp.concatenate(es, axis=1)


def _layer1a(h1, accs1, deg, p):
    sts = ('c', 'co', 'sl', 'ip')
    return pl.pallas_call(
        _l1a_body,
        grid=(NB,),
        in_specs=[_nb(D)] * 4 + [_ab(D)] * 4 + [_nb(1)]
                 + [_full((D, H))] * 8 + [_full((1, H))] * 4
                 + [_full((H, H)), _full((H, 1)), _full((1, 1))],
        out_specs=[_nb(H)] * 4 + [_nb(4)],
        out_shape=[jax.ShapeDtypeStruct((N, H), jnp.float32)] * 4
                  + [jax.ShapeDtypeStruct((N, 4), jnp.float32)],
    )(*h1, *accs1, deg,
      *[p['sage_%s_1_Wself' % st] for st in sts],
      *[p['sage_%s_1_Wneigh' % st] for st in sts],
      *[p['sage_%s_1_b' % st][None, :] for st in sts],
      p['attn_lin_W'], p['attn_W'], p['attn_b'][None, :])


def _l1b_body(hs, as_, deg_ref, lc, lco, lsl, lip, e4_ref,
              wss, wns, bs, alin, aw, ab_, fwt, fwb, fcb,
              p_ref, q_ref, a128_ref):
    inv = 1.0 / deg_ref[...]
    h2 = _sage_one(hs[...], as_[...], inv, wss, wns, bs)
    lin_s = _dot(h2, alin[...])
    e_s = _dot(lin_s, aw[...]) + ab_[...]
    e4 = e4_ref[...]
    lins = [lin_s, lc[...], lco[...], lsl[...], lip[...]]
    es = [e_s] + [e4[:, t:t + 1] for t in range(4)]
    m = es[0]
    for t in range(1, 5):
        m = jnp.maximum(m, es[t])
    exps = [jnp.exp(e - m) for e in es]
    z = exps[0] + exps[1] + exps[2] + exps[3] + exps[4]
    inv_z = 1.0 / z
    a = [ex * inv_z for ex in exps]
    hfin = jnp.zeros((BN, H), jnp.float32)
    for t in range(5):
        hfin = hfin + lins[t] * a[t]
    p_ref[...] = _dot(hfin, fwt[...]) + fcb[...]
    q_ref[...] = _dot(hfin, fwb[...])
    a128_ref[...] = jnp.concatenate(
        a + [jnp.zeros((BN, 123), jnp.float32)], axis=1)


def _layer1b(h1_s, acc1_s, deg, lins, e4, p):
    return pl.pallas_call(
        _l1b_body,
        grid=(NB,),
        in_specs=[_nb(D), _ab(D), _nb(1)] + [_nb(D)] * 4 + [_nb(4)]
                 + [_full((D, H)), _full((D, H)), _full((1, H)),
                    _full((H, H)), _full((H, 1)), _full((1, 1)),
                    _full((H, H)), _full((H, H)), _full((1, H))],
        out_specs=[_nb(H), _nb(H), _nb(H)],
        out_shape=[jax.ShapeDtypeStruct((N, H), jnp.float32)] * 3,
    )(h1_s, acc1_s, deg, *lins, e4,
      p['sage_s_1_Wself'], p['sage_s_1_Wneigh'], p['sage_s_1_b'][None, :],
      p['attn_lin_W'], p['attn_W'], p['attn_b'][None, :],
      p['fc_W'][:H], p['fc_W'][H:], p['fc_b'][None, :])


def _edge_mlp_body(gp_ref, gq_ref, wot, bo, out_ref):
    t = _leaky(gp_ref[...] + gq_ref[...])
    out_ref[...] = lax.dot_general(
        wot[...], t, (((1,), (1,)), ((), ())),
        preferred_element_type=jnp.float32) + bo[...]


def _edge_mlp(gp, gq, p, be):
    ne = gp.shape[0]
    eb = pl.BlockSpec((be, D), lambda i: (i, 0))
    score_t = pl.pallas_call(
        _edge_mlp_body,
        grid=(ne // be,),
        in_specs=[eb, eb, _full((2, D)), _full((2, 1))],
        out_specs=pl.BlockSpec((2, be), lambda i: (0, i)),
        out_shape=jax.ShapeDtypeStruct((2, ne), jnp.float32),
    )(gp, gq, p['fc_out_W'].T, p['fc_out_b'][:, None])
    return score_t


def kernel(inputs_s, inputs_sm, inputs_c, inputs_co, inputs_sl, inputs_ip,
           edge_index, edge_index_sub, params):
    p = params
    vec_cat, vec_co, vec_sl, ip_aug = _emb(
        inputs_c, inputs_co, inputs_sl, inputs_ip, p)

    src3 = edge_index[0].reshape(NW * NCH, CB, AB)
    dst3 = edge_index[1].reshape(NW * NCH, CB, AB)
    zeros_h = jnp.zeros((N2, D), jnp.float32)

    accs0a = _agg([vec_cat, vec_co, vec_sl, ip_aug], src3, dst3, zeros_h)
    vec_url = _lstm(inputs_s, p)
    accs0b = _agg([vec_url], src3, dst3, zeros_h)

    *h1a, deg = _layer0a(vec_cat, vec_co, vec_sl, inputs_ip, accs0a, p)
    h1_s = _layer0b(vec_url, accs0b[0], deg, p)

    accs1a = _agg(h1a, src3, dst3, zeros_h)
    accs1b = _agg([h1_s], src3, dst3, zeros_h)

    *lins, e4 = _layer1a(h1a, accs1a, deg, p)
    P, Q, attn128 = _layer1b(h1_s, accs1b[0], deg, lins, e4, p)

    se, de = edge_index_sub[0], edge_index_sub[1]
    gp1, gq1, ga1 = _edge_gather_half(P, Q, attn128, se[:E1], de[:E1], NB1)
    gp2, gq2, ga2 = _edge_gather_half(P, Q, attn128, se[E1:], de[E1:], NB2)
    st1 = _edge_mlp(gp1, gq1, p, 5120)
    st2 = _edge_mlp(gp2, gq2, p, 2560)

    score = jnp.concatenate([st1.T, st2.T], axis=0)
    a5 = jnp.concatenate([ga1[:, :5], ga2[:, :5]], axis=0)
    attn_out = jnp.concatenate([a5[:, None, :], a5[:, None, :]], axis=1)
    return score, attn_out


# speedup vs baseline: 1.3826x; 1.0676x over previous
import functools

import jax
import jax.numpy as jnp
from jax import lax
from jax.experimental import pallas as pl
from jax.experimental.pallas import tpu as pltpu
from jax.experimental.pallas import tpu_sc as plsc

N = 10000
E = 320000
D = 128
H = 128
L = 20
SLOPE = 0.01

NB = 10
BN = N // NB


def _leaky(x):
    return jnp.where(x >= 0, x, SLOPE * x)


def _dot(a, b):
    return jnp.dot(a, b, preferred_element_type=jnp.float32)


def _full(shape):
    return pl.BlockSpec(shape, lambda i: (0,) * len(shape))


def _nb(w):
    return pl.BlockSpec((BN, w), lambda i: (i, 0))


def _ab(w):
    return pl.BlockSpec((2, BN, w), lambda i: (0, i, 0))


def _blk3(w):
    return pl.BlockSpec((1, BN, w), lambda i: (i, 0, 0))


def _onehot(idx_col):
    io = lax.broadcasted_iota(jnp.int32, (BN, 128), 1)
    return (idx_col == io).astype(jnp.float32)


def _emb_body(c_ref, co_ref, sl_ref, ip_ref, ecat_ref, eco_ref, esl_ref,
              vc_ref, vco_ref, vsl_ref, ipa_ref):
    vc_ref[...] = _dot(_onehot(c_ref[0]), ecat_ref[...])
    vco_ref[...] = _dot(_onehot(co_ref[0]), eco_ref[...])
    vsl_ref[...] = _dot(_onehot(sl_ref[0]), esl_ref[...])
    ipa_ref[...] = jnp.concatenate(
        [ip_ref[...], jnp.ones((BN, 1), jnp.float32),
         jnp.zeros((BN, 95), jnp.float32)], axis=1)


def _emb(inputs_c, inputs_co, inputs_sl, inputs_ip, p):
    ecat = jnp.zeros((128, D), jnp.float32).at[:101].set(p['emb_cat'])
    eco = jnp.zeros((128, D), jnp.float32).at[:92].set(p['emb_co'])
    esl = jnp.zeros((128, D), jnp.float32).at[:6].set(p['emb_sl'])
    return pl.pallas_call(
        _emb_body,
        grid=(NB,),
        in_specs=[_blk3(1), _blk3(1), _blk3(1), _nb(32),
                  _full((128, D)), _full((128, D)), _full((128, D))],
        out_specs=[_nb(D)] * 4,
        out_shape=[jax.ShapeDtypeStruct((N, D), jnp.float32)] * 4,
    )(inputs_c.reshape(NB, BN, 1), inputs_co.reshape(NB, BN, 1),
      inputs_sl.reshape(NB, BN, 1), inputs_ip, ecat, eco, esl)


def _lstm_body(s_ref, t2f_ref, t2b_ref, whf_ref, whb_ref, bf_ref, bb_ref,
               fcw_ref, fcb_ref, vu_ref):
    hf = jnp.zeros((BN, H), jnp.float32)
    cf = jnp.zeros((BN, H), jnp.float32)
    hb = jnp.zeros((BN, H), jnp.float32)
    cb = jnp.zeros((BN, H), jnp.float32)

    def step(h, c, oh, t2, wh, bias):
        g = _dot(oh, t2) + _dot(h, wh) + bias
        i = jax.nn.sigmoid(g[:, :H])
        f = jax.nn.sigmoid(g[:, H:2 * H])
        gg = jnp.tanh(g[:, 2 * H:3 * H])
        o = jax.nn.sigmoid(g[:, 3 * H:])
        c2 = f * c + i * gg
        h2 = o * jnp.tanh(c2)
        return h2, c2

    for t in range(L):
        ohf = _onehot(s_ref[0, :, t:t + 1])
        ohb = _onehot(s_ref[0, :, L - 1 - t:L - t])
        hf, cf = step(hf, cf, ohf, t2f_ref[...], whf_ref[...], bf_ref[...])
        hb, cb = step(hb, cb, ohb, t2b_ref[...], whb_ref[...], bb_ref[...])

    hcat = jnp.concatenate([hf, hb], axis=1)
    vu_ref[...] = _leaky(_dot(hcat, fcw_ref[...]) + fcb_ref[...])


def _lstm(inputs_s, p):
    t2f = _dot(p['emb_url'], p['lstm_Wih_f'])
    t2b = _dot(p['emb_url'], p['lstm_Wih_b'])
    bf = (p['lstm_bih_f'] + p['lstm_bhh_f'])[None, :]
    bb = (p['lstm_bih_b'] + p['lstm_bhh_b'])[None, :]
    return pl.pallas_call(
        _lstm_body,
        grid=(NB,),
        in_specs=[_blk3(L),
                  _full((D, 4 * H)), _full((D, 4 * H)),
                  _full((H, 4 * H)), _full((H, 4 * H)),
                  _full((1, 4 * H)), _full((1, 4 * H)),
                  _full((2 * H, D)), _full((1, D))],
        out_specs=_nb(D),
        out_shape=jax.ShapeDtypeStruct((N, D), jnp.float32),
    )(inputs_s.reshape(NB, BN, L), t2f, t2b, p['lstm_Whh_f'],
      p['lstm_Whh_b'], bf, bb, p['fc_lstm_W'], p['fc_lstm_b'][None, :])


NW = 32
EPW = E // NW
AB = 125
NCH = 2
CB = 40
N2 = 10240
NPS = N2 // 16


@functools.lru_cache(maxsize=None)
def _make_agg(nt):
    mesh = plsc.VectorSubcoreMesh(core_axis_name="c", subcore_axis_name="s")
    scratch = [pltpu.VMEM((CB, AB), jnp.int32),
               pltpu.VMEM((CB, AB), jnp.int32),
               pltpu.SemaphoreType.DMA, pltpu.SemaphoreType.DMA,
               pltpu.VMEM_SHARED((N2, D), jnp.float32),
               pltpu.VMEM((AB, D), jnp.float32),
               pltpu.VMEM((AB, D), jnp.float32)]

    def body(*refs):
        hs = refs[:nt]
        src3, dst3, zeros_h = refs[nt], refs[nt + 1], refs[nt + 2]
        outs = refs[nt + 3:2 * nt + 3]
        src_v, dst_v, sem0, sem1, acc, b0, b1 = refs[2 * nt + 3:]
        c = lax.axis_index("c")
        s = lax.axis_index("s")
        wid = c * 16 + s

        for t in range(nt):
            h = hs[t]
            pltpu.sync_copy(zeros_h.at[pl.ds(s * NPS, NPS)],
                            acc.at[pl.ds(s * NPS, NPS)])
            plsc.subcore_barrier()

            @pl.loop(0, NCH)
            def _(ch, h=h):
                pltpu.sync_copy(src3.at[wid * NCH + ch], src_v)
                pltpu.sync_copy(dst3.at[wid * NCH + ch], dst_v)
                pltpu.async_copy(h.at[src_v.at[0]], b0, sem0)
                pltpu.async_copy(h.at[src_v.at[1]], b1, sem1)

                @pl.loop(0, CB, step=2)
                def _(j, h=h):
                    for bi, (buf, sem) in enumerate(((b0, sem0), (b1, sem1))):
                        k = j + bi
                        pltpu.make_async_copy(h.at[src_v.at[k]], buf,
                                              sem).wait()
                        pltpu.sync_copy(buf, acc.at[dst_v.at[k]], add=True)

                        @pl.when(k + 2 < CB)
                        def _(h=h, buf=buf, sem=sem, k=k):
                            pltpu.async_copy(h.at[src_v.at[k + 2]], buf, sem)

            plsc.subcore_barrier()
            pltpu.sync_copy(acc.at[pl.ds(s * NPS, NPS)], outs[t].at[wid])

    out_type = [jax.ShapeDtypeStruct((NW, NPS, D), jnp.float32)
                for _ in range(nt)]
    return pl.kernel(body, out_type=out_type, mesh=mesh,
                     scratch_types=scratch)


def _agg(h_list, src3, dst3, zeros_h):
    outs = _make_agg(len(h_list))(*h_list, src3, dst3, zeros_h)
    if not isinstance(outs, (list, tuple)):
        outs = (outs,)
    return [o.reshape(2, N2, D) for o in outs]


GB = 80


@functools.lru_cache(maxsize=None)
def _make_edge_gather(nbat):
    epw = nbat * GB
    mesh = plsc.VectorSubcoreMesh(core_axis_name="c", subcore_axis_name="s")
    scratch = [pltpu.VMEM((nbat, GB), jnp.int32),
               pltpu.VMEM((nbat, GB), jnp.int32)]
    for _ in range(2):
        scratch += [pltpu.VMEM((GB, D), jnp.float32),
                    pltpu.VMEM((GB, D), jnp.float32),
                    pltpu.VMEM((GB, D), jnp.float32)]
    scratch += [pltpu.SemaphoreType.DMA] * 6

    def body(P, Q, A16, se3, de3, gp_o, gq_o, ga_o, se_v, de_v,
             p0, q0, a0, p1, q1, a1, *gs):
        c = lax.axis_index("c")
        s = lax.axis_index("s")
        wid = c * 16 + s
        pltpu.sync_copy(se3.at[wid], se_v)
        pltpu.sync_copy(de3.at[wid], de_v)
        slots = ((p0, q0, a0), (p1, q1, a1))

        def gath(k, sl):
            bp, bq, ba = slots[sl]
            pltpu.async_copy(P.at[se_v.at[k]], bp, gs[3 * sl])
            pltpu.async_copy(Q.at[de_v.at[k]], bq, gs[3 * sl + 1])
            pltpu.async_copy(A16.at[se_v.at[k]], ba, gs[3 * sl + 2])

        def drain_and_write(k, sl, prefetch):
            bp, bq, ba = slots[sl]
            base = wid * epw + k * GB
            pltpu.make_async_copy(P.at[se_v.at[k]], bp, gs[3 * sl]).wait()
            pltpu.make_async_copy(Q.at[de_v.at[k]], bq,
                                  gs[3 * sl + 1]).wait()
            pltpu.make_async_copy(A16.at[se_v.at[k]], ba,
                                  gs[3 * sl + 2]).wait()
            pltpu.sync_copy(bp, gp_o.at[pl.ds(base, GB)])
            pltpu.sync_copy(bq, gq_o.at[pl.ds(base, GB)])
            pltpu.sync_copy(ba, ga_o.at[pl.ds(base, GB)])
            if prefetch:
                @pl.when(k + 2 < nbat)
                def _(k=k, sl=sl):
                    gath(k + 2, sl)

        gath(0, 0)
        gath(1, 1)

        if nbat % 2 == 0:
            @pl.loop(0, nbat, step=2)
            def _(j):
                for sl in range(2):
                    drain_and_write(j + sl, sl, True)
        else:
            @pl.loop(0, nbat - 1, step=2)
            def _(j):
                for sl in range(2):
                    drain_and_write(j + sl, sl, True)

            drain_and_write(nbat - 1, (nbat - 1) % 2, False)

    out_type = [jax.ShapeDtypeStruct((NW * epw, D), jnp.float32)] * 3
    return pl.kernel(body, out_type=out_type, mesh=mesh,
                     scratch_types=scratch)


def _edge_gather_half(P, Q, attn128, se, de, nbat):
    se3 = se.reshape(NW, nbat, GB)
    de3 = de.reshape(NW, nbat, GB)
    return _make_edge_gather(nbat)(P, Q, attn128, se3, de3)


E1 = 163840
NB1 = E1 // (NW * GB)
NB2 = (E - E1) // (NW * GB)


def _sage_one(h, acc, inv, ws, wn, b):
    mean = (acc[0] + acc[1]) * inv
    return _leaky(_dot(h, ws[...]) + _dot(mean, wn[...]) + b[...])


def _l0a_body(hc_ref, hco_ref, hsl_ref, hip_ref,
              ac_ref, aco_ref, asl_ref, aip_ref,
              wsc, wnc, wsco, wnco, wssl, wnsl, wsip, wnip,
              bc, bco, bsl, bip,
              oc_ref, oco_ref, osl_ref, oip_ref, deg_ref):
    deg = jnp.maximum(aip_ref[0, :, 32:33] + aip_ref[1, :, 32:33], 1.0)
    inv = 1.0 / deg
    deg_ref[...] = deg
    oc_ref[...] = _sage_one(hc_ref[...], ac_ref[...], inv, wsc, wnc, bc)
    oco_ref[...] = _sage_one(hco_ref[...], aco_ref[...], inv, wsco, wnco, bco)
    osl_ref[...] = _sage_one(hsl_ref[...], asl_ref[...], inv, wssl, wnsl, bsl)
    mean_ip = (aip_ref[0, :, :32] + aip_ref[1, :, :32]) * inv
    oip_ref[...] = _leaky(_dot(hip_ref[...], wsip[...]) +
                          _dot(mean_ip, wnip[...]) + bip[...])


def _layer0a(h_c, h_co, h_sl, h_ip, accs, p):
    w = lambda st: (p['sage_%s_0_Wself' % st], p['sage_%s_0_Wneigh' % st])
    b = lambda st: p['sage_%s_0_b' % st][None, :]
    return pl.pallas_call(
        _l0a_body,
        grid=(NB,),
        in_specs=[_nb(D), _nb(D), _nb(D), _nb(32),
                  _ab(D), _ab(D), _ab(D), _ab(D)]
                 + [_full((D, H))] * 6 + [_full((32, H))] * 2
                 + [_full((1, H))] * 4,
        out_specs=[_nb(H)] * 4 + [_nb(1)],
        out_shape=[jax.ShapeDtypeStruct((N, H), jnp.float32)] * 4
                  + [jax.ShapeDtypeStruct((N, 1), jnp.float32)],
    )(h_c, h_co, h_sl, h_ip, *accs,
      *w('c'), *w('co'), *w('sl'), *w('ip'), b('c'), b('co'), b('sl'),
      b('ip'))


def _l0b_body(hs_ref, as_ref, deg_ref, wss, wns, bs, os_ref):
    inv = 1.0 / deg_ref[...]
    os_ref[...] = _sage_one(hs_ref[...], as_ref[...], inv, wss, wns, bs)


def _layer0b(h_s, acc_s, deg, p):
    return pl.pallas_call(
        _l0b_body,
        grid=(NB,),
        in_specs=[_nb(D), _ab(D), _nb(1), _full((D, H)), _full((D, H)),
                  _full((1, H))],
        out_specs=_nb(H),
        out_shape=jax.ShapeDtypeStruct((N, H), jnp.float32),
    )(h_s, acc_s, deg, p['sage_s_0_Wself'], p['sage_s_0_Wneigh'],
      p['sage_s_0_b'][None, :])


def _l1a_body(hc, hco, hsl, hip, ac, aco, asl, aip, deg_ref,
              ws0, ws1, ws2, ws3, wn0, wn1, wn2, wn3, b0, b1, b2, b3,
              alin, aw, ab_,
              lc_ref, lco_ref, lsl_ref, lip_ref, e4_ref):
    inv = 1.0 / deg_ref[...]
    es = []
    for h, a, ws_, wn_, b_, o_ref in (
            (hc, ac, ws0, wn0, b0, lc_ref),
            (hco, aco, ws1, wn1, b1, lco_ref),
            (hsl, asl, ws2, wn2, b2, lsl_ref),
            (hip, aip, ws3, wn3, b3, lip_ref)):
        h2 = _sage_one(h[...], a[...], inv, ws_, wn_, b_)
        lin = _dot(h2, alin[...])
        o_ref[...] = lin
        es.append(_dot(lin, aw[...]) + ab_[...])
    e4_ref[...] = jnp.concatenate(es, axis=1)


def _layer1a(h1, accs1, deg, p):
    sts = ('c', 'co', 'sl', 'ip')
    return pl.pallas_call(
        _l1a_body,
        grid=(NB,),
        in_specs=[_nb(D)] * 4 + [_ab(D)] * 4 + [_nb(1)]
                 + [_full((D, H))] * 8 + [_full((1, H))] * 4
                 + [_full((H, H)), _full((H, 1)), _full((1, 1))],
        out_specs=[_nb(H)] * 4 + [_nb(4)],
        out_shape=[jax.ShapeDtypeStruct((N, H), jnp.float32)] * 4
                  + [jax.ShapeDtypeStruct((N, 4), jnp.float32)],
    )(*h1, *accs1, deg,
      *[p['sage_%s_1_Wself' % st] for st in sts],
      *[p['sage_%s_1_Wneigh' % st] for st in sts],
      *[p['sage_%s_1_b' % st][None, :] for st in sts],
      p['attn_lin_W'], p['attn_W'], p['attn_b'][None, :])


def _l1b_body(hs, as_, deg_ref, lc, lco, lsl, lip, e4_ref,
              wss, wns, bs, alin, aw, ab_, fwt, fwb, fcb,
              p_ref, q_ref, a128_ref):
    inv = 1.0 / deg_ref[...]
    h2 = _sage_one(hs[...], as_[...], inv, wss, wns, bs)
    lin_s = _dot(h2, alin[...])
    e_s = _dot(lin_s, aw[...]) + ab_[...]
    e4 = e4_ref[...]
    lins = [lin_s, lc[...], lco[...], lsl[...], lip[...]]
    es = [e_s] + [e4[:, t:t + 1] for t in range(4)]
    m = es[0]
    for t in range(1, 5):
        m = jnp.maximum(m, es[t])
    exps = [jnp.exp(e - m) for e in es]
    z = exps[0] + exps[1] + exps[2] + exps[3] + exps[4]
    inv_z = 1.0 / z
    a = [ex * inv_z for ex in exps]
    hfin = jnp.zeros((BN, H), jnp.float32)
    for t in range(5):
        hfin = hfin + lins[t] * a[t]
    p_ref[...] = _dot(hfin, fwt[...]) + fcb[...]
    q_ref[...] = _dot(hfin, fwb[...])
    a128_ref[...] = jnp.concatenate(
        a + [jnp.zeros((BN, 123), jnp.float32)], axis=1)


def _layer1b(h1_s, acc1_s, deg, lins, e4, p):
    return pl.pallas_call(
        _l1b_body,
        grid=(NB,),
        in_specs=[_nb(D), _ab(D), _nb(1)] + [_nb(D)] * 4 + [_nb(4)]
                 + [_full((D, H)), _full((D, H)), _full((1, H)),
                    _full((H, H)), _full((H, 1)), _full((1, 1)),
                    _full((H, H)), _full((H, H)), _full((1, H))],
        out_specs=[_nb(H), _nb(H), _nb(H)],
        out_shape=[jax.ShapeDtypeStruct((N, H), jnp.float32)] * 3,
    )(h1_s, acc1_s, deg, *lins, e4,
      p['sage_s_1_Wself'], p['sage_s_1_Wneigh'], p['sage_s_1_b'][None, :],
      p['attn_lin_W'], p['attn_W'], p['attn_b'][None, :],
      p['fc_W'][:H], p['fc_W'][H:], p['fc_b'][None, :])


def _edge_mlp_body(gp_ref, gq_ref, wot, bo, out_ref):
    t = _leaky(gp_ref[...] + gq_ref[...])
    out_ref[...] = lax.dot_general(
        wot[...], t, (((1,), (1,)), ((), ())),
        preferred_element_type=jnp.float32) + bo[...]


def _edge_mlp(gp, gq, p, be):
    ne = gp.shape[0]
    eb = pl.BlockSpec((be, D), lambda i: (i, 0))
    score_t = pl.pallas_call(
        _edge_mlp_body,
        grid=(ne // be,),
        in_specs=[eb, eb, _full((2, D)), _full((2, 1))],
        out_specs=pl.BlockSpec((2, be), lambda i: (0, i)),
        out_shape=jax.ShapeDtypeStruct((2, ne), jnp.float32),
    )(gp, gq, p['fc_out_W'].T, p['fc_out_b'][:, None])
    return score_t


def kernel(inputs_s, inputs_sm, inputs_c, inputs_co, inputs_sl, inputs_ip,
           edge_index, edge_index_sub, params):
    p = params
    vec_cat, vec_co, vec_sl, ip_aug = _emb(
        inputs_c, inputs_co, inputs_sl, inputs_ip, p)

    src3 = edge_index[0].reshape(NW * NCH, CB, AB)
    dst3 = edge_index[1].reshape(NW * NCH, CB, AB)
    zeros_h = jnp.zeros((N2, D), jnp.float32)

    accs0a = _agg([vec_cat, vec_co, vec_sl, ip_aug], src3, dst3, zeros_h)
    vec_url = _lstm(inputs_s, p)
    accs0b = _agg([vec_url], src3, dst3, zeros_h)

    *h1a, deg = _layer0a(vec_cat, vec_co, vec_sl, inputs_ip, accs0a, p)
    h1_s = _layer0b(vec_url, accs0b[0], deg, p)

    accs1a = _agg(h1a, src3, dst3, zeros_h)
    accs1b = _agg([h1_s], src3, dst3, zeros_h)

    *lins, e4 = _layer1a(h1a, accs1a, deg, p)
    P, Q, attn128 = _layer1b(h1_s, accs1b[0], deg, lins, e4, p)

    se, de = edge_index_sub[0], edge_index_sub[1]
    gp1, gq1, ga1 = _edge_gather_half(P, Q, attn128, se[:E1], de[:E1], NB1)
    gp2, gq2, ga2 = _edge_gather_half(P, Q, attn128, se[E1:], de[E1:], NB2)
    st1 = _edge_mlp(gp1, gq1, p, 5120)
    st2 = _edge_mlp(gp2, gq2, p, 2560)

    score = jnp.concatenate([st1.T, st2.T], axis=0)
    a5 = jnp.concatenate([ga1[:, :5], ga2[:, :5]], axis=0)
    attn_out = jnp.concatenate([a5[:, None, :], a5[:, None, :]], axis=1)
    return score, attn_out


# speedup vs baseline: 1.3872x; 1.0033x over previous
import functools

import jax
import jax.numpy as jnp
from jax import lax
from jax.experimental import pallas as pl
from jax.experimental.pallas import tpu as pltpu
from jax.experimental.pallas import tpu_sc as plsc

N = 10000
E = 320000
D = 128
H = 128
L = 20
SLOPE = 0.01

NB = 10
BN = N // NB


def _leaky(x):
    return jnp.where(x >= 0, x, SLOPE * x)


def _dot(a, b):
    return jnp.dot(a, b, preferred_element_type=jnp.float32)


def _full(shape):
    return pl.BlockSpec(shape, lambda i: (0,) * len(shape))


def _nb(w):
    return pl.BlockSpec((BN, w), lambda i: (i, 0))


def _ab(w):
    return pl.BlockSpec((2, BN, w), lambda i: (0, i, 0))


def _blk3(w):
    return pl.BlockSpec((1, BN, w), lambda i: (i, 0, 0))


def _onehot(idx_col):
    io = lax.broadcasted_iota(jnp.int32, (BN, 128), 1)
    return (idx_col == io).astype(jnp.float32)


def _emb_body(c_ref, co_ref, sl_ref, ip_ref, ecat_ref, eco_ref, esl_ref,
              vc_ref, vco_ref, vsl_ref, ipa_ref):
    vc_ref[...] = _dot(_onehot(c_ref[0]), ecat_ref[...])
    vco_ref[...] = _dot(_onehot(co_ref[0]), eco_ref[...])
    vsl_ref[...] = _dot(_onehot(sl_ref[0]), esl_ref[...])
    ipa_ref[...] = jnp.concatenate(
        [ip_ref[...], jnp.ones((BN, 1), jnp.float32),
         jnp.zeros((BN, 95), jnp.float32)], axis=1)


def _emb(inputs_c, inputs_co, inputs_sl, inputs_ip, p):
    ecat = jnp.zeros((128, D), jnp.float32).at[:101].set(p['emb_cat'])
    eco = jnp.zeros((128, D), jnp.float32).at[:92].set(p['emb_co'])
    esl = jnp.zeros((128, D), jnp.float32).at[:6].set(p['emb_sl'])
    return pl.pallas_call(
        _emb_body,
        grid=(NB,),
        in_specs=[_blk3(1), _blk3(1), _blk3(1), _nb(32),
                  _full((128, D)), _full((128, D)), _full((128, D))],
        out_specs=[_nb(D)] * 4,
        out_shape=[jax.ShapeDtypeStruct((N, D), jnp.float32)] * 4,
    )(inputs_c.reshape(NB, BN, 1), inputs_co.reshape(NB, BN, 1),
      inputs_sl.reshape(NB, BN, 1), inputs_ip, ecat, eco, esl)


def _lstm_body(s_ref, t2f_ref, t2b_ref, whf_ref, whb_ref, bf_ref, bb_ref,
               fcw_ref, fcb_ref, vu_ref):
    hf = jnp.zeros((BN, H), jnp.float32)
    cf = jnp.zeros((BN, H), jnp.float32)
    hb = jnp.zeros((BN, H), jnp.float32)
    cb = jnp.zeros((BN, H), jnp.float32)

    def step(h, c, oh, t2, wh, bias):
        g = _dot(oh, t2) + _dot(h, wh) + bias
        i = jax.nn.sigmoid(g[:, :H])
        f = jax.nn.sigmoid(g[:, H:2 * H])
        gg = jnp.tanh(g[:, 2 * H:3 * H])
        o = jax.nn.sigmoid(g[:, 3 * H:])
        c2 = f * c + i * gg
        h2 = o * jnp.tanh(c2)
        return h2, c2

    for t in range(L):
        ohf = _onehot(s_ref[0, :, t:t + 1])
        ohb = _onehot(s_ref[0, :, L - 1 - t:L - t])
        hf, cf = step(hf, cf, ohf, t2f_ref[...], whf_ref[...], bf_ref[...])
        hb, cb = step(hb, cb, ohb, t2b_ref[...], whb_ref[...], bb_ref[...])

    hcat = jnp.concatenate([hf, hb], axis=1)
    vu_ref[...] = _leaky(_dot(hcat, fcw_ref[...]) + fcb_ref[...])


def _lstm(inputs_s, p):
    t2f = _dot(p['emb_url'], p['lstm_Wih_f'])
    t2b = _dot(p['emb_url'], p['lstm_Wih_b'])
    bf = (p['lstm_bih_f'] + p['lstm_bhh_f'])[None, :]
    bb = (p['lstm_bih_b'] + p['lstm_bhh_b'])[None, :]
    return pl.pallas_call(
        _lstm_body,
        grid=(NB,),
        in_specs=[_blk3(L),
                  _full((D, 4 * H)), _full((D, 4 * H)),
                  _full((H, 4 * H)), _full((H, 4 * H)),
                  _full((1, 4 * H)), _full((1, 4 * H)),
                  _full((2 * H, D)), _full((1, D))],
        out_specs=_nb(D),
        out_shape=jax.ShapeDtypeStruct((N, D), jnp.float32),
    )(inputs_s.reshape(NB, BN, L), t2f, t2b, p['lstm_Whh_f'],
      p['lstm_Whh_b'], bf, bb, p['fc_lstm_W'], p['fc_lstm_b'][None, :])


NW = 32
EPW = E // NW
AB = 125
NCH = 2
CB = 40
N2 = 10240
NPS = N2 // 16


@functools.lru_cache(maxsize=None)
def _make_agg(nt):
    mesh = plsc.VectorSubcoreMesh(core_axis_name="c", subcore_axis_name="s")
    scratch = [pltpu.VMEM((CB, AB), jnp.int32),
               pltpu.VMEM((CB, AB), jnp.int32),
               pltpu.SemaphoreType.DMA, pltpu.SemaphoreType.DMA,
               pltpu.VMEM_SHARED((N2, D), jnp.float32),
               pltpu.VMEM((AB, D), jnp.float32),
               pltpu.VMEM((AB, D), jnp.float32)]

    def body(*refs):
        hs = refs[:nt]
        src3, dst3, zeros_h = refs[nt], refs[nt + 1], refs[nt + 2]
        outs = refs[nt + 3:2 * nt + 3]
        src_v, dst_v, sem0, sem1, acc, b0, b1 = refs[2 * nt + 3:]
        c = lax.axis_index("c")
        s = lax.axis_index("s")
        wid = c * 16 + s

        for t in range(nt):
            h = hs[t]
            pltpu.sync_copy(zeros_h.at[pl.ds(s * NPS, NPS)],
                            acc.at[pl.ds(s * NPS, NPS)])
            plsc.subcore_barrier()

            @pl.loop(0, NCH)
            def _(ch, h=h):
                pltpu.sync_copy(src3.at[wid * NCH + ch], src_v)
                pltpu.sync_copy(dst3.at[wid * NCH + ch], dst_v)
                pltpu.async_copy(h.at[src_v.at[0]], b0, sem0)
                pltpu.async_copy(h.at[src_v.at[1]], b1, sem1)

                @pl.loop(0, CB, step=2)
                def _(j, h=h):
                    for bi, (buf, sem) in enumerate(((b0, sem0), (b1, sem1))):
                        k = j + bi
                        pltpu.make_async_copy(h.at[src_v.at[k]], buf,
                                              sem).wait()
                        pltpu.sync_copy(buf, acc.at[dst_v.at[k]], add=True)

                        @pl.when(k + 2 < CB)
                        def _(h=h, buf=buf, sem=sem, k=k):
                            pltpu.async_copy(h.at[src_v.at[k + 2]], buf, sem)

            plsc.subcore_barrier()
            pltpu.sync_copy(acc.at[pl.ds(s * NPS, NPS)], outs[t].at[wid])

    out_type = [jax.ShapeDtypeStruct((NW, NPS, D), jnp.float32)
                for _ in range(nt)]
    return pl.kernel(body, out_type=out_type, mesh=mesh,
                     scratch_types=scratch)


def _agg(h_list, src3, dst3, zeros_h):
    outs = _make_agg(len(h_list))(*h_list, src3, dst3, zeros_h)
    if not isinstance(outs, (list, tuple)):
        outs = (outs,)
    return [o.reshape(2, N2, D) for o in outs]


GB = 80


@functools.lru_cache(maxsize=None)
def _make_edge_gather(nbat):
    epw = nbat * GB
    mesh = plsc.VectorSubcoreMesh(core_axis_name="c", subcore_axis_name="s")
    scratch = [pltpu.VMEM((nbat, GB), jnp.int32),
               pltpu.VMEM((nbat, GB), jnp.int32)]
    for _ in range(2):
        scratch += [pltpu.VMEM((GB, D), jnp.float32),
                    pltpu.VMEM((GB, D), jnp.float32),
                    pltpu.VMEM((GB, D), jnp.float32)]
    scratch += [pltpu.SemaphoreType.DMA] * 6

    def body(P, Q, A16, se3, de3, gp_o, gq_o, ga_o, se_v, de_v,
             p0, q0, a0, p1, q1, a1, *gs):
        c = lax.axis_index("c")
        s = lax.axis_index("s")
        wid = c * 16 + s
        pltpu.sync_copy(se3.at[wid], se_v)
        pltpu.sync_copy(de3.at[wid], de_v)
        slots = ((p0, q0, a0), (p1, q1, a1))

        def gath(k, sl):
            bp, bq, ba = slots[sl]
            pltpu.async_copy(P.at[se_v.at[k]], bp, gs[3 * sl])
            pltpu.async_copy(Q.at[de_v.at[k]], bq, gs[3 * sl + 1])
            pltpu.async_copy(A16.at[se_v.at[k]], ba, gs[3 * sl + 2])

        def drain_and_write(k, sl, prefetch):
            bp, bq, ba = slots[sl]
            base = wid * epw + k * GB
            pltpu.make_async_copy(P.at[se_v.at[k]], bp, gs[3 * sl]).wait()
            pltpu.make_async_copy(Q.at[de_v.at[k]], bq,
                                  gs[3 * sl + 1]).wait()
            pltpu.make_async_copy(A16.at[se_v.at[k]], ba,
                                  gs[3 * sl + 2]).wait()
            pltpu.sync_copy(bp, gp_o.at[pl.ds(base, GB)])
            pltpu.sync_copy(bq, gq_o.at[pl.ds(base, GB)])
            pltpu.sync_copy(ba, ga_o.at[pl.ds(base, GB)])
            if prefetch:
                @pl.when(k + 2 < nbat)
                def _(k=k, sl=sl):
                    gath(k + 2, sl)

        gath(0, 0)
        gath(1, 1)

        if nbat % 2 == 0:
            @pl.loop(0, nbat, step=2)
            def _(j):
                for sl in range(2):
                    drain_and_write(j + sl, sl, True)
        else:
            @pl.loop(0, nbat - 1, step=2)
            def _(j):
                for sl in range(2):
                    drain_and_write(j + sl, sl, True)

            drain_and_write(nbat - 1, (nbat - 1) % 2, False)

    out_type = [jax.ShapeDtypeStruct((NW * epw, D), jnp.float32)] * 3
    return pl.kernel(body, out_type=out_type, mesh=mesh,
                     scratch_types=scratch)


def _edge_gather_half(P, Q, attn128, se, de, nbat):
    se3 = se.reshape(NW, nbat, GB)
    de3 = de.reshape(NW, nbat, GB)
    return _make_edge_gather(nbat)(P, Q, attn128, se3, de3)


E1 = 163840
NB1 = E1 // (NW * GB)
NB2 = (E - E1) // (NW * GB)


def _sage_one(h, acc, inv, ws, wn, b):
    mean = (acc[0] + acc[1]) * inv
    return _leaky(_dot(h, ws[...]) + _dot(mean, wn[...]) + b[...])


def _l0a_body(hc_ref, hco_ref, hsl_ref, hip_ref,
              ac_ref, aco_ref, asl_ref, aip_ref,
              wsc, wnc, wsco, wnco, wssl, wnsl, wsip, wnip,
              bc, bco, bsl, bip,
              oc_ref, oco_ref, osl_ref, oip_ref, deg_ref):
    deg = jnp.maximum(aip_ref[0, :, 32:33] + aip_ref[1, :, 32:33], 1.0)
    inv = 1.0 / deg
    deg_ref[...] = deg
    oc_ref[...] = _sage_one(hc_ref[...], ac_ref[...], inv, wsc, wnc, bc)
    oco_ref[...] = _sage_one(hco_ref[...], aco_ref[...], inv, wsco, wnco, bco)
    osl_ref[...] = _sage_one(hsl_ref[...], asl_ref[...], inv, wssl, wnsl, bsl)
    mean_ip = (aip_ref[0, :, :32] + aip_ref[1, :, :32]) * inv
    oip_ref[...] = _leaky(_dot(hip_ref[...], wsip[...]) +
                          _dot(mean_ip, wnip[...]) + bip[...])


def _layer0a(h_c, h_co, h_sl, h_ip, accs, p):
    w = lambda st: (p['sage_%s_0_Wself' % st], p['sage_%s_0_Wneigh' % st])
    b = lambda st: p['sage_%s_0_b' % st][None, :]
    return pl.pallas_call(
        _l0a_body,
        grid=(NB,),
        in_specs=[_nb(D), _nb(D), _nb(D), _nb(32),
                  _ab(D), _ab(D), _ab(D), _ab(D)]
                 + [_full((D, H))] * 6 + [_full((32, H))] * 2
                 + [_full((1, H))] * 4,
        out_specs=[_nb(H)] * 4 + [_nb(1)],
        out_shape=[jax.ShapeDtypeStruct((N, H), jnp.float32)] * 4
                  + [jax.ShapeDtypeStruct((N, 1), jnp.float32)],
    )(h_c, h_co, h_sl, h_ip, *accs,
      *w('c'), *w('co'), *w('sl'), *w('ip'), b('c'), b('co'), b('sl'),
      b('ip'))


def _l0b_body(hs_ref, as_ref, deg_ref, wss, wns, bs, os_ref):
    inv = 1.0 / deg_ref[...]
    os_ref[...] = _sage_one(hs_ref[...], as_ref[...], inv, wss, wns, bs)


def _layer0b(h_s, acc_s, deg, p):
    return pl.pallas_call(
        _l0b_body,
        grid=(NB,),
        in_specs=[_nb(D), _ab(D), _nb(1), _full((D, H)), _full((D, H)),
                  _full((1, H))],
        out_specs=_nb(H),
        out_shape=jax.ShapeDtypeStruct((N, H), jnp.float32),
    )(h_s, acc_s, deg, p['sage_s_0_Wself'], p['sage_s_0_Wneigh'],
      p['sage_s_0_b'][None, :])


def _l1a_body(hc, hco, hsl, hip, ac, aco, asl, aip, deg_ref,
              ws0, ws1, ws2, ws3, wn0, wn1, wn2, wn3, b0, b1, b2, b3,
              alin, aw, ab_,
              lc_ref, lco_ref, lsl_ref, lip_ref, e4_ref):
    inv = 1.0 / deg_ref[...]
    es = []
    for h, a, ws_, wn_, b_, o_ref in (
            (hc, ac, ws0, wn0, b0, lc_ref),
            (hco, aco, ws1, wn1, b1, lco_ref),
            (hsl, asl, ws2, wn2, b2, lsl_ref),
            (hip, aip, ws3, wn3, b3, lip_ref)):
        h2 = _sage_one(h[...], a[...], inv, ws_, wn_, b_)
        lin = _dot(h2, alin[...])
        o_ref[...] = lin
        es.append(_dot(lin, aw[...]) + ab_[...])
    e4_ref[...] = jnp.concatenate(es, axis=1)


def _layer1a(h1, accs1, deg, p):
    sts = ('c', 'co', 'sl', 'ip')
    return pl.pallas_call(
        _l1a_body,
        grid=(NB,),
        in_specs=[_nb(D)] * 4 + [_ab(D)] * 4 + [_nb(1)]
                 + [_full((D, H))] * 8 + [_full((1, H))] * 4
                 + [_full((H, H)), _full((H, 1)), _full((1, 1))],
        out_specs=[_nb(H)] * 4 + [_nb(4)],
        out_shape=[jax.ShapeDtypeStruct((N, H), jnp.float32)] * 4
                  + [jax.ShapeDtypeStruct((N, 4), jnp.float32)],
    )(*h1, *accs1, deg,
      *[p['sage_%s_1_Wself' % st] for st in sts],
      *[p['sage_%s_1_Wneigh' % st] for st in sts],
      *[p['sage_%s_1_b' % st][None, :] for st in sts],
      p['attn_lin_W'], p['attn_W'], p['attn_b'][None, :])


def _l1b_body(hs, as_, deg_ref, lc, lco, lsl, lip, e4_ref,
              wss, wns, bs, alin, aw, ab_, fwt, fwb, fcb,
              p_ref, q_ref, a128_ref):
    inv = 1.0 / deg_ref[...]
    h2 = _sage_one(hs[...], as_[...], inv, wss, wns, bs)
    lin_s = _dot(h2, alin[...])
    e_s = _dot(lin_s, aw[...]) + ab_[...]
    e4 = e4_ref[...]
    lins = [lin_s, lc[...], lco[...], lsl[...], lip[...]]
    es = [e_s] + [e4[:, t:t + 1] for t in range(4)]
    m = es[0]
    for t in range(1, 5):
        m = jnp.maximum(m, es[t])
    exps = [jnp.exp(e - m) for e in es]
    z = exps[0] + exps[1] + exps[2] + exps[3] + exps[4]
    inv_z = 1.0 / z
    a = [ex * inv_z for ex in exps]
    hfin = jnp.zeros((BN, H), jnp.float32)
    for t in range(5):
        hfin = hfin + lins[t] * a[t]
    p_ref[...] = _dot(hfin, fwt[...]) + fcb[...]
    q_ref[...] = _dot(hfin, fwb[...])
    a128_ref[...] = jnp.concatenate(
        a + [jnp.zeros((BN, 123), jnp.float32)], axis=1)


def _layer1b(h1_s, acc1_s, deg, lins, e4, p):
    return pl.pallas_call(
        _l1b_body,
        grid=(NB,),
        in_specs=[_nb(D), _ab(D), _nb(1)] + [_nb(D)] * 4 + [_nb(4)]
                 + [_full((D, H)), _full((D, H)), _full((1, H)),
                    _full((H, H)), _full((H, 1)), _full((1, 1)),
                    _full((H, H)), _full((H, H)), _full((1, H))],
        out_specs=[_nb(H), _nb(H), _nb(H)],
        out_shape=[jax.ShapeDtypeStruct((N, H), jnp.float32)] * 3,
    )(h1_s, acc1_s, deg, *lins, e4,
      p['sage_s_1_Wself'], p['sage_s_1_Wneigh'], p['sage_s_1_b'][None, :],
      p['attn_lin_W'], p['attn_W'], p['attn_b'][None, :],
      p['fc_W'][:H], p['fc_W'][H:], p['fc_b'][None, :])


def _edge_mlp_body(gp_ref, gq_ref, wot, bo, out_ref):
    t = _leaky(gp_ref[...] + gq_ref[...])
    out_ref[...] = lax.dot_general(
        wot[...], t, (((1,), (1,)), ((), ())),
        preferred_element_type=jnp.float32) + bo[...]


def _edge_mlp(gp, gq, p, be):
    ne = gp.shape[0]
    eb = pl.BlockSpec((be, D), lambda i: (i, 0))
    score_t = pl.pallas_call(
        _edge_mlp_body,
        grid=(ne // be,),
        in_specs=[eb, eb, _full((2, D)), _full((2, 1))],
        out_specs=pl.BlockSpec((2, be), lambda i: (0, i)),
        out_shape=jax.ShapeDtypeStruct((2, ne), jnp.float32),
    )(gp, gq, p['fc_out_W'].T, p['fc_out_b'][:, None])
    return score_t


def kernel(inputs_s, inputs_sm, inputs_c, inputs_co, inputs_sl, inputs_ip,
           edge_index, edge_index_sub, params):
    p = params
    vec_cat, vec_co, vec_sl, ip_aug = _emb(
        inputs_c, inputs_co, inputs_sl, inputs_ip, p)

    src3 = edge_index[0].reshape(NW * NCH, CB, AB)
    dst3 = edge_index[1].reshape(NW * NCH, CB, AB)
    zeros_h = jnp.zeros((N2, D), jnp.float32)

    accs0a = _agg([vec_cat, vec_co, vec_sl, ip_aug], src3, dst3, zeros_h)
    vec_url = _lstm(inputs_s, p)
    accs0b = _agg([vec_url], src3, dst3, zeros_h)

    *h1a, deg = _layer0a(vec_cat, vec_co, vec_sl, inputs_ip, accs0a, p)
    h1_s = _layer0b(vec_url, accs0b[0], deg, p)

    accs1a = _agg(h1a, src3, dst3, zeros_h)
    accs1b = _agg([h1_s], src3, dst3, zeros_h)

    *lins, e4 = _layer1a(h1a, accs1a, deg, p)
    P, Q, attn128 = _layer1b(h1_s, accs1b[0], deg, lins, e4, p)

    se, de = edge_index_sub[0], edge_index_sub[1]
    gp1, gq1, ga1 = _edge_gather_half(P, Q, attn128, se[:E1], de[:E1], NB1)
    P2, Q2, attn2, gp1, gq1, ga1 = lax.optimization_barrier(
        (P, Q, attn128, gp1, gq1, ga1))
    gp2, gq2, ga2 = _edge_gather_half(P2, Q2, attn2, se[E1:], de[E1:], NB2)
    st1 = _edge_mlp(gp1, gq1, p, 5120)
    st2 = _edge_mlp(gp2, gq2, p, 2560)

    score = jnp.concatenate([st1.T, st2.T], axis=0)
    a5 = jnp.concatenate([ga1[:, :5], ga2[:, :5]], axis=0)
    attn_out = jnp.concatenate([a5[:, None, :], a5[:, None, :]], axis=1)
    return score, attn_out
